# baseline (speedup 1.0000x reference)
import jax
import jax.numpy as jnp
import numpy as np
from jax.experimental import pallas as pl
from jax.experimental.pallas import tpu as pltpu

C = 2048
A = 24
N = C * A
HID = 128
LAT = 256
RBF = 16
CUT = 6.0
MAXZ = 100
LAYERS = 2

CB = 8
BA = CB * A
NB = C // CB

_SIG2 = (CUT / RBF) ** 2
_INV2S = 1.0 / (2.0 * _SIG2)
_DELTA = CUT / (RBF - 1)
_UK = _DELTA / _SIG2
_KR = np.exp(-(2.0 * np.arange(RBF - 1) + 1.0) * _DELTA ** 2 * _INV2S)

_HI = jax.lax.Precision.HIGHEST

_S = np.zeros((CB, BA), np.float32)
for _c in range(CB):
    _S[_c, _c * A:(_c + 1) * A] = 1.0
_E24 = np.tile(np.eye(A, dtype=np.float32), (CB, 1))
_E24T = np.tile(np.eye(A, dtype=np.float32), (1, CB))
_PMASK = np.tile(1.0 - np.eye(A, dtype=np.float32), (1, CB))
_cid = np.arange(BA) // A
_BD = (_cid[:, None] == _cid[None, :]).astype(np.float32)


def _block_kernel(z_ref, fpk_ref, frow_ref, types_ref, len_ref, ang_ref,
                  S_ref, E24_ref, E24T_ref, pmask_ref, bd_ref,
                  emb_ref, Wz_ref, bz_ref, wrbf_ref, W1_ref, b1_ref,
                  wf_ref, Watom_ref, batom_ref, F_ref, logit_ref):
    f32 = jnp.float32

    ang = ang_ref[:] * (np.pi / 180.0)
    cosang = jnp.cos(ang)
    ca, cb_, cg = cosang[:, 0], cosang[:, 1], cosang[:, 2]
    sg = jnp.clip(jnp.sin(ang[:, 2]), 1e-6, None)
    ln = len_ref[:]
    a, b, c = ln[:, 0], ln[:, 1], ln[:, 2]
    cy = (ca - cb_ * cg) / sg
    cz = jnp.sqrt(jnp.clip(1.0 - cb_ ** 2 - cy ** 2, 1e-6, None))
    coefs = jnp.stack([a, b * cg, b * sg, c * cb_, c * cy, c * cz], axis=0)

    left = jnp.concatenate([fpk_ref[0], coefs], axis=0)
    ex = jnp.dot(left, S_ref[:], preferred_element_type=f32, precision=_HI)
    t1x, t1y, t1z = ex[0:A], ex[A:2 * A], ex[2 * A:3 * A]
    l00 = ex[3 * A][None, :]
    l10 = ex[3 * A + 1][None, :]
    l11 = ex[3 * A + 2][None, :]
    l20 = ex[3 * A + 3][None, :]
    l21 = ex[3 * A + 4][None, :]
    l22 = ex[3 * A + 5][None, :]

    frow = frow_ref[0]
    dx = t1x - frow[0][None, :]
    dx = dx - jnp.round(dx)
    dy = t1y - frow[1][None, :]
    dy = dy - jnp.round(dy)
    dz = t1z - frow[2][None, :]
    dz = dz - jnp.round(dz)
    cxx = dx * l00 + dy * l10 + dz * l20
    cyy = dy * l11 + dz * l21
    czz = dz * l22
    d2 = cxx * cxx + cyy * cyy + czz * czz + 1e-8
    inv_d = jax.lax.rsqrt(d2)
    dc = jnp.minimum(d2 * inv_d, CUT)

    env = 1.0 - dc * (1.0 / CUT)
    env = env * env * pmask_ref[:]

    e = jnp.exp(dc * dc * (-_INV2S))
    u = jnp.exp(dc * _UK)
    w0 = e * wrbf_ref[0, 0]
    w1 = e * wrbf_ref[1, 0]
    for r in range(RBF - 1):
        e = (e * u) * _KR[r]
        w0 = w0 + e * wrbf_ref[0, r + 1]
        w1 = w1 + e * wrbf_ref[1, r + 1]
    w0 = w0 * env
    w1 = w1 * env

    t = jnp.clip(types_ref[0, 0, :] - 1, 0, MAXZ - 1)
    oh = (t[:, None] == jax.lax.broadcasted_iota(jnp.int32, (BA, MAXZ), 1)
          ).astype(f32)
    Hemb = jnp.dot(oh, emb_ref[:], preferred_element_type=f32)
    Hz = jnp.dot(z_ref[:], Wz_ref[:], preferred_element_type=f32) + bz_ref[:][None, :]
    H = Hemb + jnp.broadcast_to(Hz[:, None, :], (CB, A, HID)).reshape(BA, HID)

    bd = bd_ref[:]
    for l in range(LAYERS):
        Wl = jnp.dot(E24_ref[:], w0 if l == 0 else w1,
                     preferred_element_type=f32) * bd
        m = jnp.dot(Wl, H, preferred_element_type=f32)
        H = H + jax.nn.relu(
            jnp.dot(m, W1_ref[l], preferred_element_type=f32) + b1_ref[l][None, :])

    Hw = H * wf_ref[:][None, :]
    s = jax.lax.dot_general(Hw, H, (((1,), (1,)), ((), ())),
                            preferred_element_type=f32)
    s = s * bd
    spack = jnp.dot(E24T_ref[:], s, preferred_element_type=f32,
                    precision=_HI)
    spe = spack * env
    red = (((1,), (1,)), ((), ()))
    Fx = jax.lax.dot_general(spe * (cxx * inv_d), S_ref[:], red,
                             preferred_element_type=f32, precision=_HI)
    Fy = jax.lax.dot_general(spe * (cyy * inv_d), S_ref[:], red,
                             preferred_element_type=f32, precision=_HI)
    Fz = jax.lax.dot_general(spe * (czz * inv_d), S_ref[:], red,
                             preferred_element_type=f32, precision=_HI)
    F_ref[0, 0] = Fx
    F_ref[0, 1] = Fy
    F_ref[0, 2] = Fz

    logit_ref[:] = (jnp.dot(H, Watom_ref[:], preferred_element_type=f32)
                    + batom_ref[:][None, :])


def kernel(z, pred_frac_coords, pred_atom_types, num_atoms, lengths, angles,
           atom_emb, Wz, bz, w_rbf, W1, b1, w_f, W_atom, b_atom):
    del num_atoms
    frac4 = pred_frac_coords.reshape(NB, CB, A, 3)
    fpk = frac4.transpose(0, 3, 2, 1).reshape(NB, 3 * A, CB)
    frow = frac4.transpose(0, 3, 1, 2).reshape(NB, 3, BA)
    types3 = pred_atom_types.reshape(NB, 1, BA)

    def rep(shape):
        return pl.BlockSpec(shape, lambda i: (0,) * len(shape))

    F, logits = pl.pallas_call(
        _block_kernel,
        grid=(NB,),
        in_specs=[
            pl.BlockSpec((CB, LAT), lambda i: (i, 0)),
            pl.BlockSpec((1, 3 * A, CB), lambda i: (i, 0, 0)),
            pl.BlockSpec((1, 3, BA), lambda i: (i, 0, 0)),
            pl.BlockSpec((1, 1, BA), lambda i: (i, 0, 0)),
            pl.BlockSpec((CB, 3), lambda i: (i, 0)),
            pl.BlockSpec((CB, 3), lambda i: (i, 0)),
            rep((CB, BA)),
            rep((BA, A)),
            rep((A, BA)),
            rep((A, BA)),
            rep((BA, BA)),
            rep((MAXZ, HID)),
            rep((LAT, HID)),
            rep((HID,)),
            rep((LAYERS, RBF)),
            rep((LAYERS, HID, HID)),
            rep((LAYERS, HID)),
            rep((HID,)),
            rep((HID, MAXZ)),
            rep((MAXZ,)),
        ],
        out_specs=(pl.BlockSpec((1, 3, A, CB), lambda i: (i, 0, 0, 0)),
                   pl.BlockSpec((BA, MAXZ), lambda i: (i, 0))),
        out_shape=(jax.ShapeDtypeStruct((NB, 3, A, CB), jnp.float32),
                   jax.ShapeDtypeStruct((N, MAXZ), jnp.float32)),
        compiler_params=pltpu.CompilerParams(
            dimension_semantics=("parallel",)),
    )(z, fpk, frow, types3, lengths, angles,
      jnp.asarray(_S), jnp.asarray(_E24), jnp.asarray(_E24T),
      jnp.asarray(_PMASK), jnp.asarray(_BD),
      atom_emb, Wz, bz, w_rbf, W1, b1, w_f, W_atom, b_atom)
    F = F.transpose(0, 3, 2, 1).reshape(N, 3)
    return (F, logits)


# speedup vs baseline: 1.1238x; 1.1238x over previous
import jax
import jax.numpy as jnp
import numpy as np
from jax.experimental import pallas as pl
from jax.experimental.pallas import tpu as pltpu

C = 2048
A = 24
N = C * A
HID = 128
LAT = 256
RBF = 16
CUT = 6.0
MAXZ = 100
LAYERS = 2

CB = 8
BA = CB * A
NB = C // CB
G = 2
NG = NB // G

_SIG2 = (CUT / RBF) ** 2
_INV2S = 1.0 / (2.0 * _SIG2)
_DELTA = CUT / (RBF - 1)
_UK = _DELTA / _SIG2
_KR = np.exp(-(2.0 * np.arange(RBF - 1) + 1.0) * _DELTA ** 2 * _INV2S)

_HI = jax.lax.Precision.HIGHEST

_S = np.zeros((CB, BA), np.float32)
for _c in range(CB):
    _S[_c, _c * A:(_c + 1) * A] = 1.0
_E24 = np.tile(np.eye(A, dtype=np.float32), (CB, 1))
_E24T = np.tile(np.eye(A, dtype=np.float32), (1, CB))
_PMASK = np.tile(1.0 - np.eye(A, dtype=np.float32), (1, CB))
_cid = np.arange(BA) // A
_BD = (_cid[:, None] == _cid[None, :]).astype(np.float32)


def _one_group(z, fpk, frow, t, ln, ang, S, E24, E24T, pmask, bd,
               emb, Wz, bz, wrbf, W1, b1, wf, Watom, batom):
    f32 = jnp.float32

    ang = ang * (np.pi / 180.0)
    cosang = jnp.cos(ang)
    ca, cb_, cg = cosang[:, 0], cosang[:, 1], cosang[:, 2]
    sg = jnp.clip(jnp.sin(ang[:, 2]), 1e-6, None)
    a, b, c = ln[:, 0], ln[:, 1], ln[:, 2]
    cy = (ca - cb_ * cg) / sg
    cz = jnp.sqrt(jnp.clip(1.0 - cb_ ** 2 - cy ** 2, 1e-6, None))
    coefs = jnp.stack([a, b * cg, b * sg, c * cb_, c * cy, c * cz], axis=0)

    left = jnp.concatenate([fpk, coefs], axis=0)
    ex = jnp.dot(left, S, preferred_element_type=f32, precision=_HI)
    t1x, t1y, t1z = ex[0:A], ex[A:2 * A], ex[2 * A:3 * A]
    l00 = ex[3 * A][None, :]
    l10 = ex[3 * A + 1][None, :]
    l11 = ex[3 * A + 2][None, :]
    l20 = ex[3 * A + 3][None, :]
    l21 = ex[3 * A + 4][None, :]
    l22 = ex[3 * A + 5][None, :]

    dx = t1x - frow[0][None, :]
    dx = dx - jnp.round(dx)
    dy = t1y - frow[1][None, :]
    dy = dy - jnp.round(dy)
    dz = t1z - frow[2][None, :]
    dz = dz - jnp.round(dz)
    cxx = dx * l00 + dy * l10 + dz * l20
    cyy = dy * l11 + dz * l21
    czz = dz * l22
    d2 = cxx * cxx + cyy * cyy + czz * czz + 1e-8
    inv_d = jax.lax.rsqrt(d2)
    dc = jnp.minimum(d2 * inv_d, CUT)

    env = 1.0 - dc * (1.0 / CUT)
    env = env * env * pmask

    e = jnp.exp(dc * dc * (-_INV2S))
    u = jnp.exp(dc * _UK)
    w0 = e * wrbf[0, 0]
    w1 = e * wrbf[1, 0]
    for r in range(RBF - 1):
        e = (e * u) * _KR[r]
        w0 = w0 + e * wrbf[0, r + 1]
        w1 = w1 + e * wrbf[1, r + 1]
    w0 = w0 * env
    w1 = w1 * env

    t = jnp.clip(t - 1, 0, MAXZ - 1)
    oh = (t[:, None] == jax.lax.broadcasted_iota(jnp.int32, (BA, MAXZ), 1)
          ).astype(f32)
    Hemb = jnp.dot(oh, emb, preferred_element_type=f32)
    Hz = jnp.dot(z, Wz, preferred_element_type=f32) + bz[None, :]
    H = Hemb + jnp.broadcast_to(Hz[:, None, :], (CB, A, HID)).reshape(BA, HID)

    for l in range(LAYERS):
        Wl = jnp.dot(E24, w0 if l == 0 else w1,
                     preferred_element_type=f32) * bd
        m = jnp.dot(Wl, H, preferred_element_type=f32)
        H = H + jax.nn.relu(
            jnp.dot(m, W1[l], preferred_element_type=f32) + b1[l][None, :])

    Hw = H * wf[None, :]
    s = jax.lax.dot_general(Hw, H, (((1,), (1,)), ((), ())),
                            preferred_element_type=f32)
    s = s * bd
    spack = jnp.dot(E24T, s, preferred_element_type=f32, precision=_HI)
    spe = spack * env
    P = jnp.concatenate([spe * (cxx * inv_d),
                         spe * (cyy * inv_d),
                         spe * (czz * inv_d)], axis=0)
    Fall = jax.lax.dot_general(P, S, (((1,), (1,)), ((), ())),
                               preferred_element_type=f32, precision=_HI)
    logits = jnp.dot(H, Watom, preferred_element_type=f32) + batom[None, :]
    return Fall.reshape(3, A, CB), logits


def _block_kernel(z_ref, fpk_ref, frow_ref, types_ref, len_ref, ang_ref,
                  S_ref, E24_ref, E24T_ref, pmask_ref, bd_ref,
                  emb_ref, Wz_ref, bz_ref, wrbf_ref, W1_ref, b1_ref,
                  wf_ref, Watom_ref, batom_ref, F_ref, logit_ref):
    S = S_ref[:]
    E24 = E24_ref[:]
    E24T = E24T_ref[:]
    pmask = pmask_ref[:]
    bd = bd_ref[:]
    emb = emb_ref[:]
    Wz = Wz_ref[:]
    bz = bz_ref[:]
    wrbf = wrbf_ref[:]
    W1 = W1_ref[:]
    b1 = b1_ref[:]
    wf = wf_ref[:]
    Watom = Watom_ref[:]
    batom = batom_ref[:]
    for g in range(G):
        Fall, logits = _one_group(
            z_ref[g * CB:(g + 1) * CB], fpk_ref[g], frow_ref[g],
            types_ref[g, 0], len_ref[g * CB:(g + 1) * CB],
            ang_ref[g * CB:(g + 1) * CB],
            S, E24, E24T, pmask, bd, emb, Wz, bz, wrbf, W1, b1, wf,
            Watom, batom)
        F_ref[g] = Fall
        logit_ref[g * BA:(g + 1) * BA] = logits


def kernel(z, pred_frac_coords, pred_atom_types, num_atoms, lengths, angles,
           atom_emb, Wz, bz, w_rbf, W1, b1, w_f, W_atom, b_atom):
    del num_atoms
    frac4 = pred_frac_coords.reshape(NB, CB, A, 3)
    fpk = frac4.transpose(0, 3, 2, 1).reshape(NB, 3 * A, CB)
    frow = frac4.transpose(0, 3, 1, 2).reshape(NB, 3, BA)
    types3 = pred_atom_types.reshape(NB, 1, BA)

    def rep(shape):
        return pl.BlockSpec(shape, lambda i: (0,) * len(shape))

    F, logits = pl.pallas_call(
        _block_kernel,
        grid=(NG,),
        in_specs=[
            pl.BlockSpec((G * CB, LAT), lambda i: (i, 0)),
            pl.BlockSpec((G, 3 * A, CB), lambda i: (i, 0, 0)),
            pl.BlockSpec((G, 3, BA), lambda i: (i, 0, 0)),
            pl.BlockSpec((G, 1, BA), lambda i: (i, 0, 0)),
            pl.BlockSpec((G * CB, 3), lambda i: (i, 0)),
            pl.BlockSpec((G * CB, 3), lambda i: (i, 0)),
            rep((CB, BA)),
            rep((BA, A)),
            rep((A, BA)),
            rep((A, BA)),
            rep((BA, BA)),
            rep((MAXZ, HID)),
            rep((LAT, HID)),
            rep((HID,)),
            rep((LAYERS, RBF)),
            rep((LAYERS, HID, HID)),
            rep((LAYERS, HID)),
            rep((HID,)),
            rep((HID, MAXZ)),
            rep((MAXZ,)),
        ],
        out_specs=(pl.BlockSpec((G, 3, A, CB), lambda i: (i, 0, 0, 0)),
                   pl.BlockSpec((G * BA, MAXZ), lambda i: (i, 0))),
        out_shape=(jax.ShapeDtypeStruct((NB, 3, A, CB), jnp.float32),
                   jax.ShapeDtypeStruct((N, MAXZ), jnp.float32)),
        compiler_params=pltpu.CompilerParams(
            dimension_semantics=("parallel",)),
    )(z, fpk, frow, types3, lengths, angles,
      jnp.asarray(_S), jnp.asarray(_E24), jnp.asarray(_E24T),
      jnp.asarray(_PMASK), jnp.asarray(_BD),
      atom_emb, Wz, bz, w_rbf, W1, b1, w_f, W_atom, b_atom)
    F = F.transpose(0, 3, 2, 1).reshape(N, 3)
    return (F, logits)


# speedup vs baseline: 1.1998x; 1.0676x over previous
import jax
import jax.numpy as jnp
import numpy as np
from jax.experimental import pallas as pl
from jax.experimental.pallas import tpu as pltpu

C = 2048
A = 24
N = C * A
HID = 128
LAT = 256
RBF = 16
CUT = 6.0
MAXZ = 100
LAYERS = 2

CB = 8
BA = CB * A
NB = C // CB
G = 4
NG = NB // G

_SIG2 = (CUT / RBF) ** 2
_INV2S = 1.0 / (2.0 * _SIG2)
_DELTA = CUT / (RBF - 1)
_UK = _DELTA / _SIG2
_KR = np.exp(-(2.0 * np.arange(RBF - 1) + 1.0) * _DELTA ** 2 * _INV2S)

_HI = jax.lax.Precision.HIGHEST

_S = np.zeros((CB, BA), np.float32)
for _c in range(CB):
    _S[_c, _c * A:(_c + 1) * A] = 1.0
_E24 = np.tile(np.eye(A, dtype=np.float32), (CB, 1))
_E24T = np.tile(np.eye(A, dtype=np.float32), (1, CB))
_PMASK = np.tile(1.0 - np.eye(A, dtype=np.float32), (1, CB))
_cid = np.arange(BA) // A
_BD = (_cid[:, None] == _cid[None, :]).astype(np.float32)


def _one_group(z, fpk, frow, t, ln, ang, S, E24, E24T, pmask, bd,
               emb, Wz, bz, wrbf, W1, b1, wf, Watom, batom):
    f32 = jnp.float32

    ang = ang * (np.pi / 180.0)
    cosang = jnp.cos(ang)
    ca, cb_, cg = cosang[:, 0], cosang[:, 1], cosang[:, 2]
    sg = jnp.clip(jnp.sin(ang[:, 2]), 1e-6, None)
    a, b, c = ln[:, 0], ln[:, 1], ln[:, 2]
    cy = (ca - cb_ * cg) / sg
    cz = jnp.sqrt(jnp.clip(1.0 - cb_ ** 2 - cy ** 2, 1e-6, None))
    coefs = jnp.stack([a, b * cg, b * sg, c * cb_, c * cy, c * cz], axis=0)

    left = jnp.concatenate([fpk, coefs], axis=0)
    ex = jnp.dot(left, S, preferred_element_type=f32, precision=_HI)
    t1x, t1y, t1z = ex[0:A], ex[A:2 * A], ex[2 * A:3 * A]
    l00 = ex[3 * A][None, :]
    l10 = ex[3 * A + 1][None, :]
    l11 = ex[3 * A + 2][None, :]
    l20 = ex[3 * A + 3][None, :]
    l21 = ex[3 * A + 4][None, :]
    l22 = ex[3 * A + 5][None, :]

    dx = t1x - frow[0][None, :]
    dx = dx - jnp.round(dx)
    dy = t1y - frow[1][None, :]
    dy = dy - jnp.round(dy)
    dz = t1z - frow[2][None, :]
    dz = dz - jnp.round(dz)
    cxx = dx * l00 + dy * l10 + dz * l20
    cyy = dy * l11 + dz * l21
    czz = dz * l22
    d2 = cxx * cxx + cyy * cyy + czz * czz + 1e-8
    inv_d = jax.lax.rsqrt(d2)
    dc = jnp.minimum(d2 * inv_d, CUT)

    env = 1.0 - dc * (1.0 / CUT)
    env = env * env * pmask

    e = jnp.exp(dc * dc * (-_INV2S))
    u = jnp.exp(dc * _UK)
    w0 = e * wrbf[0, 0]
    w1 = e * wrbf[1, 0]
    for r in range(RBF - 1):
        e = (e * u) * _KR[r]
        w0 = w0 + e * wrbf[0, r + 1]
        w1 = w1 + e * wrbf[1, r + 1]
    w0 = w0 * env
    w1 = w1 * env

    t = jnp.clip(t - 1, 0, MAXZ - 1)
    oh = (t[:, None] == jax.lax.broadcasted_iota(jnp.int32, (BA, MAXZ), 1)
          ).astype(f32)
    Hemb = jnp.dot(oh, emb, preferred_element_type=f32)
    Hz = jnp.dot(z, Wz, preferred_element_type=f32) + bz[None, :]
    H = Hemb + jnp.broadcast_to(Hz[:, None, :], (CB, A, HID)).reshape(BA, HID)

    for l in range(LAYERS):
        Wl = jnp.dot(E24, w0 if l == 0 else w1,
                     preferred_element_type=f32) * bd
        m = jnp.dot(Wl, H, preferred_element_type=f32)
        H = H + jax.nn.relu(
            jnp.dot(m, W1[l], preferred_element_type=f32) + b1[l][None, :])

    Hw = H * wf[None, :]
    s = jax.lax.dot_general(Hw, H, (((1,), (1,)), ((), ())),
                            preferred_element_type=f32)
    s = s * bd
    spack = jnp.dot(E24T, s, preferred_element_type=f32, precision=_HI)
    spe = spack * env
    P = jnp.concatenate([spe * (cxx * inv_d),
                         spe * (cyy * inv_d),
                         spe * (czz * inv_d)], axis=0)
    Fall = jax.lax.dot_general(P, S, (((1,), (1,)), ((), ())),
                               preferred_element_type=f32, precision=_HI)
    logits = jnp.dot(H, Watom, preferred_element_type=f32) + batom[None, :]
    return Fall.reshape(3, A, CB), logits


def _block_kernel(z_ref, fpk_ref, frow_ref, types_ref, len_ref, ang_ref,
                  S_ref, E24_ref, E24T_ref, pmask_ref, bd_ref,
                  emb_ref, Wz_ref, bz_ref, wrbf_ref, W1_ref, b1_ref,
                  wf_ref, Watom_ref, batom_ref, F_ref, logit_ref):
    S = S_ref[:]
    E24 = E24_ref[:]
    E24T = E24T_ref[:]
    pmask = pmask_ref[:]
    bd = bd_ref[:]
    emb = emb_ref[:]
    Wz = Wz_ref[:]
    bz = bz_ref[:]
    wrbf = wrbf_ref[:]
    W1 = W1_ref[:]
    b1 = b1_ref[:]
    wf = wf_ref[:]
    Watom = Watom_ref[:]
    batom = batom_ref[:]
    for g in range(G):
        Fall, logits = _one_group(
            z_ref[g * CB:(g + 1) * CB], fpk_ref[g], frow_ref[g],
            types_ref[g, 0], len_ref[g * CB:(g + 1) * CB],
            ang_ref[g * CB:(g + 1) * CB],
            S, E24, E24T, pmask, bd, emb, Wz, bz, wrbf, W1, b1, wf,
            Watom, batom)
        F_ref[g] = Fall
        logit_ref[g * BA:(g + 1) * BA] = logits


def kernel(z, pred_frac_coords, pred_atom_types, num_atoms, lengths, angles,
           atom_emb, Wz, bz, w_rbf, W1, b1, w_f, W_atom, b_atom):
    del num_atoms
    frac4 = pred_frac_coords.reshape(NB, CB, A, 3)
    fpk = frac4.transpose(0, 3, 2, 1).reshape(NB, 3 * A, CB)
    frow = frac4.transpose(0, 3, 1, 2).reshape(NB, 3, BA)
    types3 = pred_atom_types.reshape(NB, 1, BA)

    def rep(shape):
        return pl.BlockSpec(shape, lambda i: (0,) * len(shape))

    F, logits = pl.pallas_call(
        _block_kernel,
        grid=(NG,),
        in_specs=[
            pl.BlockSpec((G * CB, LAT), lambda i: (i, 0)),
            pl.BlockSpec((G, 3 * A, CB), lambda i: (i, 0, 0)),
            pl.BlockSpec((G, 3, BA), lambda i: (i, 0, 0)),
            pl.BlockSpec((G, 1, BA), lambda i: (i, 0, 0)),
            pl.BlockSpec((G * CB, 3), lambda i: (i, 0)),
            pl.BlockSpec((G * CB, 3), lambda i: (i, 0)),
            rep((CB, BA)),
            rep((BA, A)),
            rep((A, BA)),
            rep((A, BA)),
            rep((BA, BA)),
            rep((MAXZ, HID)),
            rep((LAT, HID)),
            rep((HID,)),
            rep((LAYERS, RBF)),
            rep((LAYERS, HID, HID)),
            rep((LAYERS, HID)),
            rep((HID,)),
            rep((HID, MAXZ)),
            rep((MAXZ,)),
        ],
        out_specs=(pl.BlockSpec((G, 3, A, CB), lambda i: (i, 0, 0, 0)),
                   pl.BlockSpec((G * BA, MAXZ), lambda i: (i, 0))),
        out_shape=(jax.ShapeDtypeStruct((NB, 3, A, CB), jnp.float32),
                   jax.ShapeDtypeStruct((N, MAXZ), jnp.float32)),
        compiler_params=pltpu.CompilerParams(
            dimension_semantics=("parallel",)),
    )(z, fpk, frow, types3, lengths, angles,
      jnp.asarray(_S), jnp.asarray(_E24), jnp.asarray(_E24T),
      jnp.asarray(_PMASK), jnp.asarray(_BD),
      atom_emb, Wz, bz, w_rbf, W1, b1, w_f, W_atom, b_atom)
    F = F.transpose(0, 3, 2, 1).reshape(N, 3)
    return (F, logits)


# speedup vs baseline: 1.2492x; 1.0412x over previous
import jax
import jax.numpy as jnp
import numpy as np
from jax.experimental import pallas as pl
from jax.experimental.pallas import tpu as pltpu

C = 2048
A = 24
N = C * A
HID = 128
LAT = 256
RBF = 16
CUT = 6.0
MAXZ = 100
LAYERS = 2

CB = 8
BA = CB * A
NB = C // CB
G = 8
NG = NB // G

_SIG2 = (CUT / RBF) ** 2
_INV2S = 1.0 / (2.0 * _SIG2)
_DELTA = CUT / (RBF - 1)
_UK = _DELTA / _SIG2
_KR = np.exp(-(2.0 * np.arange(RBF - 1) + 1.0) * _DELTA ** 2 * _INV2S)

_HI = jax.lax.Precision.HIGHEST

_S = np.zeros((CB, BA), np.float32)
for _c in range(CB):
    _S[_c, _c * A:(_c + 1) * A] = 1.0
_E24 = np.tile(np.eye(A, dtype=np.float32), (CB, 1))
_E24T = np.tile(np.eye(A, dtype=np.float32), (1, CB))
_PMASK = np.tile(1.0 - np.eye(A, dtype=np.float32), (1, CB))
_cid = np.arange(BA) // A
_BD = (_cid[:, None] == _cid[None, :]).astype(np.float32)


def _one_group(z, fpk, frow, t, ln, ang, S, E24, E24T, pmask, bd,
               emb, Wz, bz, wrbf, W1, b1, wf, Watom, batom):
    f32 = jnp.float32

    ang = ang * (np.pi / 180.0)
    cosang = jnp.cos(ang)
    ca, cb_, cg = cosang[:, 0], cosang[:, 1], cosang[:, 2]
    sg = jnp.clip(jnp.sin(ang[:, 2]), 1e-6, None)
    a, b, c = ln[:, 0], ln[:, 1], ln[:, 2]
    cy = (ca - cb_ * cg) / sg
    cz = jnp.sqrt(jnp.clip(1.0 - cb_ ** 2 - cy ** 2, 1e-6, None))
    coefs = jnp.stack([a, b * cg, b * sg, c * cb_, c * cy, c * cz], axis=0)

    left = jnp.concatenate([fpk, coefs], axis=0)
    ex = jnp.dot(left, S, preferred_element_type=f32, precision=_HI)
    t1x, t1y, t1z = ex[0:A], ex[A:2 * A], ex[2 * A:3 * A]
    l00 = ex[3 * A][None, :]
    l10 = ex[3 * A + 1][None, :]
    l11 = ex[3 * A + 2][None, :]
    l20 = ex[3 * A + 3][None, :]
    l21 = ex[3 * A + 4][None, :]
    l22 = ex[3 * A + 5][None, :]

    dx = t1x - frow[0][None, :]
    dx = dx - jnp.round(dx)
    dy = t1y - frow[1][None, :]
    dy = dy - jnp.round(dy)
    dz = t1z - frow[2][None, :]
    dz = dz - jnp.round(dz)
    cxx = dx * l00 + dy * l10 + dz * l20
    cyy = dy * l11 + dz * l21
    czz = dz * l22
    d2 = cxx * cxx + cyy * cyy + czz * czz + 1e-8
    inv_d = jax.lax.rsqrt(d2)
    dc = jnp.minimum(d2 * inv_d, CUT)

    env = 1.0 - dc * (1.0 / CUT)
    env = env * env * pmask

    e = jnp.exp(dc * dc * (-_INV2S))
    u = jnp.exp(dc * _UK)
    w0 = e * wrbf[0, 0]
    w1 = e * wrbf[1, 0]
    for r in range(RBF - 1):
        e = (e * u) * _KR[r]
        w0 = w0 + e * wrbf[0, r + 1]
        w1 = w1 + e * wrbf[1, r + 1]
    w0 = w0 * env
    w1 = w1 * env

    t = jnp.clip(t - 1, 0, MAXZ - 1)
    oh = (t[:, None] == jax.lax.broadcasted_iota(jnp.int32, (BA, MAXZ), 1)
          ).astype(f32)
    Hemb = jnp.dot(oh, emb, preferred_element_type=f32)
    Hz = jnp.dot(z, Wz, preferred_element_type=f32) + bz[None, :]
    H = Hemb + jnp.broadcast_to(Hz[:, None, :], (CB, A, HID)).reshape(BA, HID)

    for l in range(LAYERS):
        Wl = jnp.dot(E24, w0 if l == 0 else w1,
                     preferred_element_type=f32) * bd
        m = jnp.dot(Wl, H, preferred_element_type=f32)
        H = H + jax.nn.relu(
            jnp.dot(m, W1[l], preferred_element_type=f32) + b1[l][None, :])

    Hw = H * wf[None, :]
    s = jax.lax.dot_general(Hw, H, (((1,), (1,)), ((), ())),
                            preferred_element_type=f32)
    s = s * bd
    spack = jnp.dot(E24T, s, preferred_element_type=f32, precision=_HI)
    spe = spack * env
    P = jnp.concatenate([spe * (cxx * inv_d),
                         spe * (cyy * inv_d),
                         spe * (czz * inv_d)], axis=0)
    Fall = jax.lax.dot_general(P, S, (((1,), (1,)), ((), ())),
                               preferred_element_type=f32, precision=_HI)
    logits = jnp.dot(H, Watom, preferred_element_type=f32) + batom[None, :]
    return Fall.reshape(3, A, CB), logits


def _block_kernel(z_ref, fpk_ref, frow_ref, types_ref, len_ref, ang_ref,
                  S_ref, E24_ref, E24T_ref, pmask_ref, bd_ref,
                  emb_ref, Wz_ref, bz_ref, wrbf_ref, W1_ref, b1_ref,
                  wf_ref, Watom_ref, batom_ref, F_ref, logit_ref):
    S = S_ref[:]
    E24 = E24_ref[:]
    E24T = E24T_ref[:]
    pmask = pmask_ref[:]
    bd = bd_ref[:]
    emb = emb_ref[:]
    Wz = Wz_ref[:]
    bz = bz_ref[:]
    wrbf = wrbf_ref[:]
    W1 = W1_ref[:]
    b1 = b1_ref[:]
    wf = wf_ref[:]
    Watom = Watom_ref[:]
    batom = batom_ref[:]
    for g in range(G):
        Fall, logits = _one_group(
            z_ref[g * CB:(g + 1) * CB], fpk_ref[g], frow_ref[g],
            types_ref[g, 0], len_ref[g * CB:(g + 1) * CB],
            ang_ref[g * CB:(g + 1) * CB],
            S, E24, E24T, pmask, bd, emb, Wz, bz, wrbf, W1, b1, wf,
            Watom, batom)
        F_ref[g] = Fall
        logit_ref[g * BA:(g + 1) * BA] = logits


def kernel(z, pred_frac_coords, pred_atom_types, num_atoms, lengths, angles,
           atom_emb, Wz, bz, w_rbf, W1, b1, w_f, W_atom, b_atom):
    del num_atoms
    frac4 = pred_frac_coords.reshape(NB, CB, A, 3)
    fpk = frac4.transpose(0, 3, 2, 1).reshape(NB, 3 * A, CB)
    frow = frac4.transpose(0, 3, 1, 2).reshape(NB, 3, BA)
    types3 = pred_atom_types.reshape(NB, 1, BA)

    def rep(shape):
        return pl.BlockSpec(shape, lambda i: (0,) * len(shape))

    F, logits = pl.pallas_call(
        _block_kernel,
        grid=(NG,),
        in_specs=[
            pl.BlockSpec((G * CB, LAT), lambda i: (i, 0)),
            pl.BlockSpec((G, 3 * A, CB), lambda i: (i, 0, 0)),
            pl.BlockSpec((G, 3, BA), lambda i: (i, 0, 0)),
            pl.BlockSpec((G, 1, BA), lambda i: (i, 0, 0)),
            pl.BlockSpec((G * CB, 3), lambda i: (i, 0)),
            pl.BlockSpec((G * CB, 3), lambda i: (i, 0)),
            rep((CB, BA)),
            rep((BA, A)),
            rep((A, BA)),
            rep((A, BA)),
            rep((BA, BA)),
            rep((MAXZ, HID)),
            rep((LAT, HID)),
            rep((HID,)),
            rep((LAYERS, RBF)),
            rep((LAYERS, HID, HID)),
            rep((LAYERS, HID)),
            rep((HID,)),
            rep((HID, MAXZ)),
            rep((MAXZ,)),
        ],
        out_specs=(pl.BlockSpec((G, 3, A, CB), lambda i: (i, 0, 0, 0)),
                   pl.BlockSpec((G * BA, MAXZ), lambda i: (i, 0))),
        out_shape=(jax.ShapeDtypeStruct((NB, 3, A, CB), jnp.float32),
                   jax.ShapeDtypeStruct((N, MAXZ), jnp.float32)),
        compiler_params=pltpu.CompilerParams(
            dimension_semantics=("parallel",)),
    )(z, fpk, frow, types3, lengths, angles,
      jnp.asarray(_S), jnp.asarray(_E24), jnp.asarray(_E24T),
      jnp.asarray(_PMASK), jnp.asarray(_BD),
      atom_emb, Wz, bz, w_rbf, W1, b1, w_f, W_atom, b_atom)
    F = F.transpose(0, 3, 2, 1).reshape(N, 3)
    return (F, logits)


# speedup vs baseline: 1.3249x; 1.0607x over previous
import jax
import jax.numpy as jnp
import numpy as np
from jax.experimental import pallas as pl
from jax.experimental.pallas import tpu as pltpu

C = 2048
A = 24
N = C * A
HID = 128
LAT = 256
RBF = 16
CUT = 6.0
MAXZ = 100
LAYERS = 2

CB = 8
BA = CB * A
G = 8
GA = G * A
CPS = G * CB
NG = C // CPS

_SIG2 = (CUT / RBF) ** 2
_INV2S = 1.0 / (2.0 * _SIG2)
_DELTA = CUT / (RBF - 1)
_UK = _DELTA / _SIG2
_KR = np.exp(-(2.0 * np.arange(RBF - 1) + 1.0) * _DELTA ** 2 * _INV2S)

_HP = jax.lax.Precision.HIGHEST

_S = np.zeros((CB, BA), np.float32)
for _c in range(CB):
    _S[_c, _c * A:(_c + 1) * A] = 1.0
_EA = np.zeros((GA, G), np.float32)
for _g in range(G):
    _EA[_g * A:(_g + 1) * A, _g] = 1.0
_E24 = np.tile(np.eye(A, dtype=np.float32), (CB, 1))
_E24T = np.tile(np.eye(A, dtype=np.float32), (1, CB))
_PMASK = np.tile(1.0 - np.eye(A, dtype=np.float32), (G, CB))
_cid = np.arange(BA) // A
_BD = (_cid[:, None] == _cid[None, :]).astype(np.float32)


def _block_kernel(z_ref, fpk_ref, fr_ref, types_ref, len_ref, ang_ref,
                  S_ref, EA_ref, E24_ref, E24T_ref, pmask_ref, bd_ref,
                  emb_ref, Wz_ref, bz_ref, wrbf_ref, W1_ref, b1_ref,
                  wf_ref, Watom_ref, batom_ref, F_ref, logit_ref):
    f32 = jnp.float32
    S = S_ref[:]
    EA = EA_ref[:]
    E24 = E24_ref[:]
    E24T = E24T_ref[:]
    bd = bd_ref[:]
    wrbf = wrbf_ref[:]

    rad = np.pi / 180.0
    ca = jnp.cos(ang_ref[0, 0] * rad)
    cb_ = jnp.cos(ang_ref[0, 1] * rad)
    gam = ang_ref[0, 2] * rad
    cg = jnp.cos(gam)
    sg = jnp.clip(jnp.sin(gam), 1e-6, None)
    a, b, c = len_ref[0, 0], len_ref[0, 1], len_ref[0, 2]
    cy = (ca - cb_ * cg) / sg
    cz = jnp.sqrt(jnp.clip(1.0 - cb_ ** 2 - cy ** 2, 1e-6, None))
    cf2 = jnp.concatenate([a, b * cg, b * sg, c * cb_, c * cy, c * cz],
                          axis=0)

    t1 = jnp.dot(fpk_ref[0], S, preferred_element_type=f32,
                 precision=_HP)
    t1x, t1y, t1z = t1[0:GA], t1[GA:2 * GA], t1[2 * GA:3 * GA]
    fr = fr_ref[0]
    t2x = jnp.dot(EA, fr[0:G], preferred_element_type=f32, precision=_HP)
    t2y = jnp.dot(EA, fr[G:2 * G], preferred_element_type=f32, precision=_HP)
    t2z = jnp.dot(EA, fr[2 * G:3 * G], preferred_element_type=f32,
                  precision=_HP)
    cfl = jnp.dot(cf2, S, preferred_element_type=f32, precision=_HP)
    l00 = jnp.dot(EA, cfl[0:G], preferred_element_type=f32, precision=_HP)
    l10 = jnp.dot(EA, cfl[G:2 * G], preferred_element_type=f32, precision=_HP)
    l11 = jnp.dot(EA, cfl[2 * G:3 * G], preferred_element_type=f32, precision=_HP)
    l20 = jnp.dot(EA, cfl[3 * G:4 * G], preferred_element_type=f32, precision=_HP)
    l21 = jnp.dot(EA, cfl[4 * G:5 * G], preferred_element_type=f32, precision=_HP)
    l22 = jnp.dot(EA, cfl[5 * G:6 * G], preferred_element_type=f32, precision=_HP)

    dx = t1x - t2x
    dx = dx - jnp.round(dx)
    dy = t1y - t2y
    dy = dy - jnp.round(dy)
    dz = t1z - t2z
    dz = dz - jnp.round(dz)
    cxx = dx * l00 + dy * l10 + dz * l20
    cyy = dy * l11 + dz * l21
    czz = dz * l22
    d2 = cxx * cxx + cyy * cyy + czz * czz + 1e-8
    inv_d = jax.lax.rsqrt(d2)
    dc = jnp.minimum(d2 * inv_d, CUT)

    env = 1.0 - dc * (1.0 / CUT)
    env = env * env * pmask_ref[:]

    e = jnp.exp(dc * dc * (-_INV2S))
    u = jnp.exp(dc * _UK)
    w0 = e * wrbf[0, 0]
    w1 = e * wrbf[1, 0]
    for r in range(RBF - 1):
        e = (e * u) * _KR[r]
        w0 = w0 + e * wrbf[0, r + 1]
        w1 = w1 + e * wrbf[1, r + 1]
    w0 = w0 * env
    w1 = w1 * env
    ux = cxx * inv_d
    uy = cyy * inv_d
    uz = czz * inv_d

    t = jnp.clip(types_ref[0, 0, :] - 1, 0, MAXZ - 1)
    oh = (t[:, None] == jax.lax.broadcasted_iota(jnp.int32, (CPS * A, MAXZ), 1)
          ).astype(f32)
    Hemb = jnp.dot(oh, emb_ref[:], preferred_element_type=f32)
    Hz = jnp.dot(z_ref[:], Wz_ref[:], preferred_element_type=f32) + bz_ref[:][None, :]
    H0 = Hemb + jnp.broadcast_to(Hz[:, None, :], (CPS, A, HID)).reshape(CPS * A, HID)

    W1w = W1_ref[:]
    b1w = b1_ref[:]
    wf = wf_ref[:]
    Watom = Watom_ref[:]
    batom = batom_ref[:]

    for g in range(G):
        rows = slice(g * A, (g + 1) * A)
        H = H0[g * BA:(g + 1) * BA]
        for l in range(LAYERS):
            wl = (w0 if l == 0 else w1)[rows]
            Wl = jnp.dot(E24, wl, preferred_element_type=f32) * bd
            m = jnp.dot(Wl, H, preferred_element_type=f32)
            H = H + jax.nn.relu(
                jnp.dot(m, W1w[l], preferred_element_type=f32) + b1w[l][None, :])

        Hw = H * wf[None, :]
        s = jax.lax.dot_general(Hw, H, (((1,), (1,)), ((), ())),
                                preferred_element_type=f32)
        s = s * bd
        spack = jnp.dot(E24T, s, preferred_element_type=f32, precision=_HP)
        spe = spack * env[rows]
        P = jnp.concatenate([spe * ux[rows], spe * uy[rows], spe * uz[rows]],
                            axis=0)
        Fall = jax.lax.dot_general(P, S, (((1,), (1,)), ((), ())),
                                   preferred_element_type=f32, precision=_HP)
        F_ref[g] = Fall.reshape(3, A, CB)
        logit_ref[g * BA:(g + 1) * BA] = (
            jnp.dot(H, Watom, preferred_element_type=f32) + batom[None, :])


def kernel(z, pred_frac_coords, pred_atom_types, num_atoms, lengths, angles,
           atom_emb, Wz, bz, w_rbf, W1, b1, w_f, W_atom, b_atom):
    del num_atoms
    frac6 = pred_frac_coords.reshape(NG, G, CB, A, 3)
    fpk = frac6.transpose(0, 4, 1, 3, 2).reshape(NG, 3 * GA, CB)
    fr = frac6.transpose(0, 4, 1, 2, 3).reshape(NG, 3 * G, BA)
    types3 = pred_atom_types.reshape(NG, 1, CPS * A)
    len4 = lengths.reshape(NG, G, CB, 3).transpose(0, 3, 1, 2)
    ang4 = angles.reshape(NG, G, CB, 3).transpose(0, 3, 1, 2)

    def rep(shape):
        return pl.BlockSpec(shape, lambda i: (0,) * len(shape))

    F, logits = pl.pallas_call(
        _block_kernel,
        grid=(NG,),
        in_specs=[
            pl.BlockSpec((CPS, LAT), lambda i: (i, 0)),
            pl.BlockSpec((1, 3 * GA, CB), lambda i: (i, 0, 0)),
            pl.BlockSpec((1, 3 * G, BA), lambda i: (i, 0, 0)),
            pl.BlockSpec((1, 1, CPS * A), lambda i: (i, 0, 0)),
            pl.BlockSpec((1, 3, G, CB), lambda i: (i, 0, 0, 0)),
            pl.BlockSpec((1, 3, G, CB), lambda i: (i, 0, 0, 0)),
            rep((CB, BA)),
            rep((GA, G)),
            rep((BA, A)),
            rep((A, BA)),
            rep((GA, BA)),
            rep((BA, BA)),
            rep((MAXZ, HID)),
            rep((LAT, HID)),
            rep((HID,)),
            rep((LAYERS, RBF)),
            rep((LAYERS, HID, HID)),
            rep((LAYERS, HID)),
            rep((HID,)),
            rep((HID, MAXZ)),
            rep((MAXZ,)),
        ],
        out_specs=(pl.BlockSpec((G, 3, A, CB), lambda i: (i, 0, 0, 0)),
                   pl.BlockSpec((CPS * A, MAXZ), lambda i: (i, 0))),
        out_shape=(jax.ShapeDtypeStruct((NG * G, 3, A, CB), jnp.float32),
                   jax.ShapeDtypeStruct((N, MAXZ), jnp.float32)),
        compiler_params=pltpu.CompilerParams(
            dimension_semantics=("parallel",)),
    )(z, fpk, fr, types3, len4, ang4,
      jnp.asarray(_S), jnp.asarray(_EA), jnp.asarray(_E24),
      jnp.asarray(_E24T), jnp.asarray(_PMASK), jnp.asarray(_BD),
      atom_emb, Wz, bz, w_rbf, W1, b1, w_f, W_atom, b_atom)
    F = F.transpose(0, 3, 2, 1).reshape(N, 3)
    return (F, logits)


# speedup vs baseline: 1.9788x; 1.4935x over previous
import jax
import jax.numpy as jnp
import numpy as np
from jax.experimental import pallas as pl
from jax.experimental.pallas import tpu as pltpu

C = 2048
A = 24
N = C * A
HID = 128
LAT = 256
RBF = 16
CUT = 6.0
MAXZ = 100
LAYERS = 2

CB = 8
BA = CB * A
G = 8
GA = G * A
CPS = G * CB
NG = C // CPS

_SIG2 = (CUT / RBF) ** 2
_INV2S = 1.0 / (2.0 * _SIG2)
_DELTA = CUT / (RBF - 1)
_UK = _DELTA / _SIG2
_KR = np.exp(-(2.0 * np.arange(RBF - 1) + 1.0) * _DELTA ** 2 * _INV2S)

_HP = jax.lax.Precision.HIGHEST

_S = np.zeros((CB, BA), np.float32)
for _c in range(CB):
    _S[_c, _c * A:(_c + 1) * A] = 1.0
_EA = np.zeros((GA, G), np.float32)
for _g in range(G):
    _EA[_g * A:(_g + 1) * A, _g] = 1.0
_E24 = np.tile(np.eye(A, dtype=np.float32), (CB, 1))
_E24T = np.tile(np.eye(A, dtype=np.float32), (1, CB))
_PMASK = np.tile(1.0 - np.eye(A, dtype=np.float32), (G, CB))
_cid = np.arange(BA) // A
_BD = (_cid[:, None] == _cid[None, :]).astype(np.float32)


def _block_kernel(z_ref, fpk_ref, fr_ref, types_ref, len_ref, ang_ref,
                  S_ref, EA_ref, E24_ref, E24T_ref, pmask_ref, bd_ref,
                  emb_ref, Wz_ref, bz_ref, wrbf_ref, W1_ref, b1_ref,
                  wf_ref, Watom_ref, batom_ref, F_ref, logit_ref):
    f32 = jnp.float32
    S = S_ref[:]
    EA = EA_ref[:]
    E24 = E24_ref[:]
    E24T = E24T_ref[:]
    bd = bd_ref[:]
    wrbf = wrbf_ref[:]

    rad = np.pi / 180.0
    ca = jnp.cos(ang_ref[0, 0] * rad)
    cb_ = jnp.cos(ang_ref[0, 1] * rad)
    gam = ang_ref[0, 2] * rad
    cg = jnp.cos(gam)
    sg = jnp.clip(jnp.sin(gam), 1e-6, None)
    a, b, c = len_ref[0, 0], len_ref[0, 1], len_ref[0, 2]
    cy = (ca - cb_ * cg) / sg
    cz = jnp.sqrt(jnp.clip(1.0 - cb_ ** 2 - cy ** 2, 1e-6, None))
    cf2 = jnp.concatenate([a, b * cg, b * sg, c * cb_, c * cy, c * cz],
                          axis=0)

    t1 = jnp.dot(fpk_ref[0], S, preferred_element_type=f32,
                 precision=_HP)
    t1x, t1y, t1z = t1[0:GA], t1[GA:2 * GA], t1[2 * GA:3 * GA]

    def grow(x):
        return jnp.broadcast_to(x[:, None, :], (G, A, BA)).reshape(GA, BA)

    fr = fr_ref[0]
    t2x = grow(fr[0:G])
    t2y = grow(fr[G:2 * G])
    t2z = grow(fr[2 * G:3 * G])
    cfl = jnp.dot(cf2, S, preferred_element_type=f32, precision=_HP)
    l00 = grow(cfl[0:G])
    l10 = grow(cfl[G:2 * G])
    l11 = grow(cfl[2 * G:3 * G])
    l20 = grow(cfl[3 * G:4 * G])
    l21 = grow(cfl[4 * G:5 * G])
    l22 = grow(cfl[5 * G:6 * G])

    dx = t1x - t2x
    dx = dx - jnp.round(dx)
    dy = t1y - t2y
    dy = dy - jnp.round(dy)
    dz = t1z - t2z
    dz = dz - jnp.round(dz)
    cxx = dx * l00 + dy * l10 + dz * l20
    cyy = dy * l11 + dz * l21
    czz = dz * l22
    d2 = cxx * cxx + cyy * cyy + czz * czz + 1e-8
    inv_d = jax.lax.rsqrt(d2)
    dc = jnp.minimum(d2 * inv_d, CUT)

    env = 1.0 - dc * (1.0 / CUT)
    env = env * env * pmask_ref[:]

    e = jnp.exp(dc * dc * (-_INV2S))
    u = jnp.exp(dc * _UK)
    w0 = e * wrbf[0, 0]
    w1 = e * wrbf[1, 0]
    for r in range(RBF - 1):
        e = (e * u) * _KR[r]
        w0 = w0 + e * wrbf[0, r + 1]
        w1 = w1 + e * wrbf[1, r + 1]
    w0 = w0 * env
    w1 = w1 * env
    ux = cxx * inv_d
    uy = cyy * inv_d
    uz = czz * inv_d

    t = jnp.clip(types_ref[0, 0, :] - 1, 0, MAXZ - 1)
    oh = (t[:, None] == jax.lax.broadcasted_iota(jnp.int32, (CPS * A, MAXZ), 1)
          ).astype(f32)
    Hemb = jnp.dot(oh, emb_ref[:], preferred_element_type=f32)
    Hz = jnp.dot(z_ref[:], Wz_ref[:], preferred_element_type=f32) + bz_ref[:][None, :]
    H0 = Hemb + jnp.broadcast_to(Hz[:, None, :], (CPS, A, HID)).reshape(CPS * A, HID)

    W1w = W1_ref[:]
    b1w = b1_ref[:]
    wf = wf_ref[:]
    Watom = Watom_ref[:]
    batom = batom_ref[:]

    spacks = []
    for g in range(G):
        rows = slice(g * A, (g + 1) * A)
        H = H0[g * BA:(g + 1) * BA]
        for l in range(LAYERS):
            wl = (w0 if l == 0 else w1)[rows]
            Wl = jnp.broadcast_to(wl[None], (CB, A, BA)).reshape(BA, BA) * bd
            m = jnp.dot(Wl, H, preferred_element_type=f32)
            H = H + jax.nn.relu(
                jnp.dot(m, W1w[l], preferred_element_type=f32) + b1w[l][None, :])

        Hw = H * wf[None, :]
        s = jax.lax.dot_general(Hw, H, (((1,), (1,)), ((), ())),
                                preferred_element_type=f32)
        s = s * bd
        spacks.append(s.reshape(CB, A, BA).sum(axis=0))
        logit_ref[g * BA:(g + 1) * BA] = (
            jnp.dot(H, Watom, preferred_element_type=f32) + batom[None, :])

    spe = jnp.concatenate(spacks, axis=0) * env
    P = jnp.concatenate([spe * ux, spe * uy, spe * uz], axis=0)
    Fall = jax.lax.dot_general(P, S, (((1,), (1,)), ((), ())),
                               preferred_element_type=f32, precision=_HP)
    for g in range(G):
        F_ref[g, 0] = Fall[g * A:(g + 1) * A]
        F_ref[g, 1] = Fall[GA + g * A:GA + (g + 1) * A]
        F_ref[g, 2] = Fall[2 * GA + g * A:2 * GA + (g + 1) * A]


def kernel(z, pred_frac_coords, pred_atom_types, num_atoms, lengths, angles,
           atom_emb, Wz, bz, w_rbf, W1, b1, w_f, W_atom, b_atom):
    del num_atoms
    frac6 = pred_frac_coords.reshape(NG, G, CB, A, 3)
    fpk = frac6.transpose(0, 4, 1, 3, 2).reshape(NG, 3 * GA, CB)
    fr = frac6.transpose(0, 4, 1, 2, 3).reshape(NG, 3 * G, BA)
    types3 = pred_atom_types.reshape(NG, 1, CPS * A)
    len4 = lengths.reshape(NG, G, CB, 3).transpose(0, 3, 1, 2)
    ang4 = angles.reshape(NG, G, CB, 3).transpose(0, 3, 1, 2)

    def rep(shape):
        return pl.BlockSpec(shape, lambda i: (0,) * len(shape))

    F, logits = pl.pallas_call(
        _block_kernel,
        grid=(NG,),
        in_specs=[
            pl.BlockSpec((CPS, LAT), lambda i: (i, 0)),
            pl.BlockSpec((1, 3 * GA, CB), lambda i: (i, 0, 0)),
            pl.BlockSpec((1, 3 * G, BA), lambda i: (i, 0, 0)),
            pl.BlockSpec((1, 1, CPS * A), lambda i: (i, 0, 0)),
            pl.BlockSpec((1, 3, G, CB), lambda i: (i, 0, 0, 0)),
            pl.BlockSpec((1, 3, G, CB), lambda i: (i, 0, 0, 0)),
            rep((CB, BA)),
            rep((GA, G)),
            rep((BA, A)),
            rep((A, BA)),
            rep((GA, BA)),
            rep((BA, BA)),
            rep((MAXZ, HID)),
            rep((LAT, HID)),
            rep((HID,)),
            rep((LAYERS, RBF)),
            rep((LAYERS, HID, HID)),
            rep((LAYERS, HID)),
            rep((HID,)),
            rep((HID, MAXZ)),
            rep((MAXZ,)),
        ],
        out_specs=(pl.BlockSpec((G, 3, A, CB), lambda i: (i, 0, 0, 0)),
                   pl.BlockSpec((CPS * A, MAXZ), lambda i: (i, 0))),
        out_shape=(jax.ShapeDtypeStruct((NG * G, 3, A, CB), jnp.float32),
                   jax.ShapeDtypeStruct((N, MAXZ), jnp.float32)),
        compiler_params=pltpu.CompilerParams(
            dimension_semantics=("parallel",)),
    )(z, fpk, fr, types3, len4, ang4,
      jnp.asarray(_S), jnp.asarray(_EA), jnp.asarray(_E24),
      jnp.asarray(_E24T), jnp.asarray(_PMASK), jnp.asarray(_BD),
      atom_emb, Wz, bz, w_rbf, W1, b1, w_f, W_atom, b_atom)
    F = F.transpose(0, 3, 2, 1).reshape(N, 3)
    return (F, logits)


# speedup vs baseline: 2.9921x; 1.5121x over previous
import jax
import jax.numpy as jnp
import numpy as np
from jax.experimental import pallas as pl
from jax.experimental.pallas import tpu as pltpu

C = 2048
A = 24
N = C * A
HID = 128
LAT = 256
RBF = 16
CUT = 6.0
MAXZ = 100
LAYERS = 2

CB = 8
BA = CB * A
G = 8
GA = G * A
CPS = G * CB
NG = C // CPS

_SIG2 = (CUT / RBF) ** 2
_INV2S = 1.0 / (2.0 * _SIG2)
_DELTA = CUT / (RBF - 1)
_UK = _DELTA / _SIG2
_KR = np.exp(-(2.0 * np.arange(RBF - 1) + 1.0) * _DELTA ** 2 * _INV2S)

_HP = jax.lax.Precision.HIGHEST

_S = np.zeros((CB, BA), np.float32)
for _c in range(CB):
    _S[_c, _c * A:(_c + 1) * A] = 1.0
_EA = np.zeros((GA, G), np.float32)
for _g in range(G):
    _EA[_g * A:(_g + 1) * A, _g] = 1.0
_E24 = np.tile(np.eye(A, dtype=np.float32), (CB, 1))
_E24T = np.tile(np.eye(A, dtype=np.float32), (1, CB))
_PMASK = np.tile(1.0 - np.eye(A, dtype=np.float32), (G, CB))
_cid = np.arange(BA) // A
_BD = (_cid[:, None] == _cid[None, :]).astype(np.float32)


def _block_kernel(z_ref, fpk_ref, fr_ref, types_ref, len_ref, ang_ref,
                  S_ref, EA_ref, E24_ref, E24T_ref, pmask_ref, bd_ref,
                  emb_ref, Wz_ref, bz_ref, wrbf_ref, W1_ref, b1_ref,
                  wf_ref, Watom_ref, batom_ref, F_ref, logit_ref):
    f32 = jnp.float32
    S = S_ref[:]
    EA = EA_ref[:]
    E24 = E24_ref[:]
    E24T = E24T_ref[:]
    bd = bd_ref[:]
    wrbf = wrbf_ref[:]

    rad = np.pi / 180.0
    ca = jnp.cos(ang_ref[0, 0] * rad)
    cb_ = jnp.cos(ang_ref[0, 1] * rad)
    gam = ang_ref[0, 2] * rad
    cg = jnp.cos(gam)
    sg = jnp.clip(jnp.sin(gam), 1e-6, None)
    a, b, c = len_ref[0, 0], len_ref[0, 1], len_ref[0, 2]
    cy = (ca - cb_ * cg) / sg
    cz = jnp.sqrt(jnp.clip(1.0 - cb_ ** 2 - cy ** 2, 1e-6, None))
    cf2 = jnp.concatenate([a, b * cg, b * sg, c * cb_, c * cy, c * cz],
                          axis=0)

    t1 = jnp.dot(fpk_ref[0], S, preferred_element_type=f32,
                 precision=_HP)
    t1x, t1y, t1z = t1[0:GA], t1[GA:2 * GA], t1[2 * GA:3 * GA]

    def grow(x):
        return jnp.broadcast_to(x[:, None, :], (G, A, BA)).reshape(GA, BA)

    fr = fr_ref[0]
    t2x = grow(fr[0:G])
    t2y = grow(fr[G:2 * G])
    t2z = grow(fr[2 * G:3 * G])
    cfl = jnp.dot(cf2, S, preferred_element_type=f32, precision=_HP)
    l00 = grow(cfl[0:G])
    l10 = grow(cfl[G:2 * G])
    l11 = grow(cfl[2 * G:3 * G])
    l20 = grow(cfl[3 * G:4 * G])
    l21 = grow(cfl[4 * G:5 * G])
    l22 = grow(cfl[5 * G:6 * G])

    dx = t1x - t2x
    dx = dx - jnp.round(dx)
    dy = t1y - t2y
    dy = dy - jnp.round(dy)
    dz = t1z - t2z
    dz = dz - jnp.round(dz)
    cxx = dx * l00 + dy * l10 + dz * l20
    cyy = dy * l11 + dz * l21
    czz = dz * l22
    d2 = cxx * cxx + cyy * cyy + czz * czz + 1e-8
    inv_d = jax.lax.rsqrt(d2)
    dc = jnp.minimum(d2 * inv_d, CUT)

    env = 1.0 - dc * (1.0 / CUT)
    env = env * env * pmask_ref[:]

    e = jnp.exp(dc * dc * (-_INV2S))
    u = jnp.exp(dc * _UK)
    w0 = e * wrbf[0, 0]
    w1 = e * wrbf[1, 0]
    for r in range(RBF - 1):
        e = (e * u) * _KR[r]
        w0 = w0 + e * wrbf[0, r + 1]
        w1 = w1 + e * wrbf[1, r + 1]
    w0 = w0 * env
    w1 = w1 * env
    ux = cxx * inv_d
    uy = cyy * inv_d
    uz = czz * inv_d

    t = jnp.clip(types_ref[0, 0, :] - 1, 0, MAXZ - 1)
    oh = (t[:, None] == jax.lax.broadcasted_iota(jnp.int32, (CPS * A, MAXZ), 1)
          ).astype(f32)
    Hemb = jnp.dot(oh, emb_ref[:], preferred_element_type=f32)
    Hz = jnp.dot(z_ref[:], Wz_ref[:], preferred_element_type=f32) + bz_ref[:][None, :]
    H0 = Hemb + jnp.broadcast_to(Hz[:, None, :], (CPS, A, HID)).reshape(CPS * A, HID)

    W1w = W1_ref[:]
    b1w = b1_ref[:]
    wf = wf_ref[:]
    Watom = Watom_ref[:]
    batom = batom_ref[:]

    H = H0
    for l in range(LAYERS):
        w = w0 if l == 0 else w1
        ms = []
        for g in range(G):
            wl = w[g * A:(g + 1) * A]
            Wl = jnp.broadcast_to(wl[None], (CB, A, BA)).reshape(BA, BA) * bd
            ms.append(jnp.dot(Wl, H[g * BA:(g + 1) * BA],
                              preferred_element_type=f32))
        m = jnp.concatenate(ms, axis=0)
        H = H + jax.nn.relu(
            jnp.dot(m, W1w[l], preferred_element_type=f32) + b1w[l][None, :])

    logit_ref[:] = jnp.dot(H, Watom, preferred_element_type=f32) + batom[None, :]

    Hw = H * wf[None, :]
    spacks = []
    for g in range(G):
        blk = slice(g * BA, (g + 1) * BA)
        s = jax.lax.dot_general(Hw[blk], H[blk], (((1,), (1,)), ((), ())),
                                preferred_element_type=f32)
        s = s * bd
        spacks.append(s.reshape(CB, A, BA).sum(axis=0))

    spe = jnp.concatenate(spacks, axis=0) * env
    P = jnp.concatenate([spe * ux, spe * uy, spe * uz], axis=0)
    Fall = jax.lax.dot_general(P, S, (((1,), (1,)), ((), ())),
                               preferred_element_type=f32, precision=_HP)
    for g in range(G):
        F_ref[g, 0] = Fall[g * A:(g + 1) * A]
        F_ref[g, 1] = Fall[GA + g * A:GA + (g + 1) * A]
        F_ref[g, 2] = Fall[2 * GA + g * A:2 * GA + (g + 1) * A]


def kernel(z, pred_frac_coords, pred_atom_types, num_atoms, lengths, angles,
           atom_emb, Wz, bz, w_rbf, W1, b1, w_f, W_atom, b_atom):
    del num_atoms
    frac6 = pred_frac_coords.reshape(NG, G, CB, A, 3)
    fpk = frac6.transpose(0, 4, 1, 3, 2).reshape(NG, 3 * GA, CB)
    fr = frac6.transpose(0, 4, 1, 2, 3).reshape(NG, 3 * G, BA)
    types3 = pred_atom_types.reshape(NG, 1, CPS * A)
    len4 = lengths.reshape(NG, G, CB, 3).transpose(0, 3, 1, 2)
    ang4 = angles.reshape(NG, G, CB, 3).transpose(0, 3, 1, 2)

    def rep(shape):
        return pl.BlockSpec(shape, lambda i: (0,) * len(shape))

    F, logits = pl.pallas_call(
        _block_kernel,
        grid=(NG,),
        in_specs=[
            pl.BlockSpec((CPS, LAT), lambda i: (i, 0)),
            pl.BlockSpec((1, 3 * GA, CB), lambda i: (i, 0, 0)),
            pl.BlockSpec((1, 3 * G, BA), lambda i: (i, 0, 0)),
            pl.BlockSpec((1, 1, CPS * A), lambda i: (i, 0, 0)),
            pl.BlockSpec((1, 3, G, CB), lambda i: (i, 0, 0, 0)),
            pl.BlockSpec((1, 3, G, CB), lambda i: (i, 0, 0, 0)),
            rep((CB, BA)),
            rep((GA, G)),
            rep((BA, A)),
            rep((A, BA)),
            rep((GA, BA)),
            rep((BA, BA)),
            rep((MAXZ, HID)),
            rep((LAT, HID)),
            rep((HID,)),
            rep((LAYERS, RBF)),
            rep((LAYERS, HID, HID)),
            rep((LAYERS, HID)),
            rep((HID,)),
            rep((HID, MAXZ)),
            rep((MAXZ,)),
        ],
        out_specs=(pl.BlockSpec((G, 3, A, CB), lambda i: (i, 0, 0, 0)),
                   pl.BlockSpec((CPS * A, MAXZ), lambda i: (i, 0))),
        out_shape=(jax.ShapeDtypeStruct((NG * G, 3, A, CB), jnp.float32),
                   jax.ShapeDtypeStruct((N, MAXZ), jnp.float32)),
        compiler_params=pltpu.CompilerParams(
            dimension_semantics=("parallel",)),
    )(z, fpk, fr, types3, len4, ang4,
      jnp.asarray(_S), jnp.asarray(_EA), jnp.asarray(_E24),
      jnp.asarray(_E24T), jnp.asarray(_PMASK), jnp.asarray(_BD),
      atom_emb, Wz, bz, w_rbf, W1, b1, w_f, W_atom, b_atom)
    F = F.transpose(0, 3, 2, 1).reshape(N, 3)
    return (F, logits)


# speedup vs baseline: 3.6039x; 1.2045x over previous
import jax
import jax.numpy as jnp
import numpy as np
from jax.experimental import pallas as pl
from jax.experimental.pallas import tpu as pltpu

C = 2048
A = 24
N = C * A
HID = 128
LAT = 256
RBF = 16
CUT = 6.0
MAXZ = 100
LAYERS = 2

CB = 8
BA = CB * A
G = 8
GA = G * A
CPS = G * CB
NG = C // CPS

_SIG2 = (CUT / RBF) ** 2
_INV2S = 1.0 / (2.0 * _SIG2)
_DELTA = CUT / (RBF - 1)
_UK = _DELTA / _SIG2
_KR = np.exp(-(2.0 * np.arange(RBF - 1) + 1.0) * _DELTA ** 2 * _INV2S)

_HP = jax.lax.Precision.HIGHEST

_S = np.zeros((CB, BA), np.float32)
for _c in range(CB):
    _S[_c, _c * A:(_c + 1) * A] = 1.0
_EA = np.zeros((GA, G), np.float32)
for _g in range(G):
    _EA[_g * A:(_g + 1) * A, _g] = 1.0
_E24 = np.tile(np.eye(A, dtype=np.float32), (CB, 1))
_E24T = np.tile(np.eye(A, dtype=np.float32), (1, CB))
_PMASK = np.tile(1.0 - np.eye(A, dtype=np.float32), (G, CB))
_cid = np.arange(BA) // A
_BD = (_cid[:, None] == _cid[None, :]).astype(np.float32)


def _dot3(x, sel_bf16, dims=None):
    f32 = jnp.float32
    x1 = x.astype(jnp.bfloat16)
    r = x - x1.astype(f32)
    x2 = r.astype(jnp.bfloat16)
    x3 = (r - x2.astype(f32)).astype(jnp.bfloat16)
    if dims is None:
        return (jnp.dot(x1, sel_bf16, preferred_element_type=f32)
                + jnp.dot(x2, sel_bf16, preferred_element_type=f32)
                + jnp.dot(x3, sel_bf16, preferred_element_type=f32))
    return (jax.lax.dot_general(x1, sel_bf16, dims, preferred_element_type=f32)
            + jax.lax.dot_general(x2, sel_bf16, dims, preferred_element_type=f32)
            + jax.lax.dot_general(x3, sel_bf16, dims, preferred_element_type=f32))


def _block_kernel(z_ref, fpk_ref, fr_ref, types_ref, len_ref, ang_ref,
                  S_ref, EA_ref, E24_ref, E24T_ref, pmask_ref, bd_ref,
                  emb_ref, Wz_ref, bz_ref, wrbf_ref, W1_ref, b1_ref,
                  wf_ref, Watom_ref, batom_ref, F_ref, logit_ref):
    f32 = jnp.float32
    S = S_ref[:]
    EA = EA_ref[:]
    E24 = E24_ref[:]
    E24T = E24T_ref[:]
    bd = bd_ref[:]
    wrbf = wrbf_ref[:]

    rad = np.pi / 180.0
    ca = jnp.cos(ang_ref[0, 0] * rad)
    cb_ = jnp.cos(ang_ref[0, 1] * rad)
    gam = ang_ref[0, 2] * rad
    cg = jnp.cos(gam)
    sg = jnp.clip(jnp.sin(gam), 1e-6, None)
    a, b, c = len_ref[0, 0], len_ref[0, 1], len_ref[0, 2]
    cy = (ca - cb_ * cg) / sg
    cz = jnp.sqrt(jnp.clip(1.0 - cb_ ** 2 - cy ** 2, 1e-6, None))
    cf2 = jnp.concatenate([a, b * cg, b * sg, c * cb_, c * cy, c * cz],
                          axis=0)

    Sb = S.astype(jnp.bfloat16)
    t1 = _dot3(fpk_ref[0], Sb)
    t1x, t1y, t1z = t1[0:GA], t1[GA:2 * GA], t1[2 * GA:3 * GA]

    def grow(x):
        return jnp.broadcast_to(x[:, None, :], (G, A, BA)).reshape(GA, BA)

    fr = fr_ref[0]
    t2x = grow(fr[0:G])
    t2y = grow(fr[G:2 * G])
    t2z = grow(fr[2 * G:3 * G])
    cfl = _dot3(cf2, Sb)
    l00 = grow(cfl[0:G])
    l10 = grow(cfl[G:2 * G])
    l11 = grow(cfl[2 * G:3 * G])
    l20 = grow(cfl[3 * G:4 * G])
    l21 = grow(cfl[4 * G:5 * G])
    l22 = grow(cfl[5 * G:6 * G])

    dx = t1x - t2x
    dx = dx - jnp.round(dx)
    dy = t1y - t2y
    dy = dy - jnp.round(dy)
    dz = t1z - t2z
    dz = dz - jnp.round(dz)
    cxx = dx * l00 + dy * l10 + dz * l20
    cyy = dy * l11 + dz * l21
    czz = dz * l22
    d2 = cxx * cxx + cyy * cyy + czz * czz + 1e-8
    inv_d = jax.lax.rsqrt(d2)
    dc = jnp.minimum(d2 * inv_d, CUT)

    env = 1.0 - dc * (1.0 / CUT)
    env = env * env * pmask_ref[:]

    e = jnp.exp(dc * dc * (-_INV2S))
    u = jnp.exp(dc * _UK)
    w0 = e * wrbf[0, 0]
    w1 = e * wrbf[1, 0]
    for r in range(RBF - 1):
        e = (e * u) * _KR[r]
        w0 = w0 + e * wrbf[0, r + 1]
        w1 = w1 + e * wrbf[1, r + 1]
    w0 = w0 * env
    w1 = w1 * env
    ux = cxx * inv_d
    uy = cyy * inv_d
    uz = czz * inv_d

    t = jnp.clip(types_ref[0, 0, :] - 1, 0, MAXZ - 1)
    oh = (t[:, None] == jax.lax.broadcasted_iota(jnp.int32, (CPS * A, MAXZ), 1)
          ).astype(f32)
    Hemb = jnp.dot(oh, emb_ref[:], preferred_element_type=f32)
    Hz = jnp.dot(z_ref[:], Wz_ref[:], preferred_element_type=f32) + bz_ref[:][None, :]
    H0 = Hemb + jnp.broadcast_to(Hz[:, None, :], (CPS, A, HID)).reshape(CPS * A, HID)

    W1w = W1_ref[:]
    b1w = b1_ref[:]
    wf = wf_ref[:]
    Watom = Watom_ref[:]
    batom = batom_ref[:]

    H = H0
    for l in range(LAYERS):
        w = w0 if l == 0 else w1
        ms = []
        for g in range(G):
            wl = w[g * A:(g + 1) * A]
            Wl = jnp.broadcast_to(wl[None], (CB, A, BA)).reshape(BA, BA) * bd
            ms.append(jnp.dot(Wl, H[g * BA:(g + 1) * BA],
                              preferred_element_type=f32))
        m = jnp.concatenate(ms, axis=0)
        H = H + jax.nn.relu(
            jnp.dot(m, W1w[l], preferred_element_type=f32) + b1w[l][None, :])

    logit_ref[:] = jnp.dot(H, Watom, preferred_element_type=f32) + batom[None, :]

    Hw = H * wf[None, :]
    spacks = []
    for g in range(G):
        blk = slice(g * BA, (g + 1) * BA)
        s = jax.lax.dot_general(Hw[blk], H[blk], (((1,), (1,)), ((), ())),
                                preferred_element_type=f32)
        s = s * bd
        spacks.append(s.reshape(CB, A, BA).sum(axis=0))

    spe = jnp.concatenate(spacks, axis=0) * env
    P = jnp.concatenate([spe * ux, spe * uy, spe * uz], axis=0)
    Fall = _dot3(P, Sb, (((1,), (1,)), ((), ())))
    for g in range(G):
        F_ref[g, 0] = Fall[g * A:(g + 1) * A]
        F_ref[g, 1] = Fall[GA + g * A:GA + (g + 1) * A]
        F_ref[g, 2] = Fall[2 * GA + g * A:2 * GA + (g + 1) * A]


def kernel(z, pred_frac_coords, pred_atom_types, num_atoms, lengths, angles,
           atom_emb, Wz, bz, w_rbf, W1, b1, w_f, W_atom, b_atom):
    del num_atoms
    frac6 = pred_frac_coords.reshape(NG, G, CB, A, 3)
    fpk = frac6.transpose(0, 4, 1, 3, 2).reshape(NG, 3 * GA, CB)
    fr = frac6.transpose(0, 4, 1, 2, 3).reshape(NG, 3 * G, BA)
    types3 = pred_atom_types.reshape(NG, 1, CPS * A)
    len4 = lengths.reshape(NG, G, CB, 3).transpose(0, 3, 1, 2)
    ang4 = angles.reshape(NG, G, CB, 3).transpose(0, 3, 1, 2)

    def rep(shape):
        return pl.BlockSpec(shape, lambda i: (0,) * len(shape))

    F, logits = pl.pallas_call(
        _block_kernel,
        grid=(NG,),
        in_specs=[
            pl.BlockSpec((CPS, LAT), lambda i: (i, 0)),
            pl.BlockSpec((1, 3 * GA, CB), lambda i: (i, 0, 0)),
            pl.BlockSpec((1, 3 * G, BA), lambda i: (i, 0, 0)),
            pl.BlockSpec((1, 1, CPS * A), lambda i: (i, 0, 0)),
            pl.BlockSpec((1, 3, G, CB), lambda i: (i, 0, 0, 0)),
            pl.BlockSpec((1, 3, G, CB), lambda i: (i, 0, 0, 0)),
            rep((CB, BA)),
            rep((GA, G)),
            rep((BA, A)),
            rep((A, BA)),
            rep((GA, BA)),
            rep((BA, BA)),
            rep((MAXZ, HID)),
            rep((LAT, HID)),
            rep((HID,)),
            rep((LAYERS, RBF)),
            rep((LAYERS, HID, HID)),
            rep((LAYERS, HID)),
            rep((HID,)),
            rep((HID, MAXZ)),
            rep((MAXZ,)),
        ],
        out_specs=(pl.BlockSpec((G, 3, A, CB), lambda i: (i, 0, 0, 0)),
                   pl.BlockSpec((CPS * A, MAXZ), lambda i: (i, 0))),
        out_shape=(jax.ShapeDtypeStruct((NG * G, 3, A, CB), jnp.float32),
                   jax.ShapeDtypeStruct((N, MAXZ), jnp.float32)),
        compiler_params=pltpu.CompilerParams(
            dimension_semantics=("parallel",)),
    )(z, fpk, fr, types3, len4, ang4,
      jnp.asarray(_S), jnp.asarray(_EA), jnp.asarray(_E24),
      jnp.asarray(_E24T), jnp.asarray(_PMASK), jnp.asarray(_BD),
      atom_emb, Wz, bz, w_rbf, W1, b1, w_f, W_atom, b_atom)
    F = F.transpose(0, 3, 2, 1).reshape(N, 3)
    return (F, logits)


# speedup vs baseline: 3.8703x; 1.0739x over previous
import jax
import jax.numpy as jnp
import numpy as np
from jax.experimental import pallas as pl
from jax.experimental.pallas import tpu as pltpu

C = 2048
A = 24
N = C * A
HID = 128
LAT = 256
RBF = 16
CUT = 6.0
MAXZ = 100
LAYERS = 2

CB = 8
BA = CB * A
G = 8
GA = G * A
CPS = G * CB
NG = C // CPS

_SIG2 = (CUT / RBF) ** 2
_INV2S = 1.0 / (2.0 * _SIG2)
_DELTA = CUT / (RBF - 1)
_UK = _DELTA / _SIG2
_KR = np.exp(-(2.0 * np.arange(RBF - 1) + 1.0) * _DELTA ** 2 * _INV2S)
_K4M = np.ones((RBF // 4, 4))
for _t in range(RBF // 4):
    for _m in range(1, 4):
        _K4M[_t, _m] = _K4M[_t, _m - 1] * _KR[4 * _t + _m - 1]
_K4C = np.array([_K4M[_t, 3] * _KR[4 * _t + 3] for _t in range(RBF // 4 - 1)])

_HP = jax.lax.Precision.HIGHEST

_S = np.zeros((CB, BA), np.float32)
for _c in range(CB):
    _S[_c, _c * A:(_c + 1) * A] = 1.0
_EA = np.zeros((GA, G), np.float32)
for _g in range(G):
    _EA[_g * A:(_g + 1) * A, _g] = 1.0
_E24 = np.tile(np.eye(A, dtype=np.float32), (CB, 1))
_E24T = np.tile(np.eye(A, dtype=np.float32), (1, CB))
_PMASK = np.tile(1.0 - np.eye(A, dtype=np.float32), (G, CB))
_cid = np.arange(BA) // A
_BD = (_cid[:, None] == _cid[None, :]).astype(np.float32)


def _dot3(x, sel_bf16, dims=None):
    f32 = jnp.float32
    x1 = x.astype(jnp.bfloat16)
    r = x - x1.astype(f32)
    x2 = r.astype(jnp.bfloat16)
    x3 = (r - x2.astype(f32)).astype(jnp.bfloat16)
    if dims is None:
        return (jnp.dot(x1, sel_bf16, preferred_element_type=f32)
                + jnp.dot(x2, sel_bf16, preferred_element_type=f32)
                + jnp.dot(x3, sel_bf16, preferred_element_type=f32))
    return (jax.lax.dot_general(x1, sel_bf16, dims, preferred_element_type=f32)
            + jax.lax.dot_general(x2, sel_bf16, dims, preferred_element_type=f32)
            + jax.lax.dot_general(x3, sel_bf16, dims, preferred_element_type=f32))


def _block_kernel(z_ref, fpk_ref, fr_ref, types_ref, len_ref, ang_ref,
                  S_ref, EA_ref, E24_ref, E24T_ref, pmask_ref, bd_ref,
                  emb_ref, Wz_ref, bz_ref, wrbf_ref, W1_ref, b1_ref,
                  wf_ref, Watom_ref, batom_ref, F_ref, logit_ref):
    f32 = jnp.float32
    S = S_ref[:]
    EA = EA_ref[:]
    E24 = E24_ref[:]
    E24T = E24T_ref[:]
    bd = bd_ref[:]
    wrbf = wrbf_ref[:]

    rad = np.pi / 180.0
    ca = jnp.cos(ang_ref[0, 0] * rad)
    cb_ = jnp.cos(ang_ref[0, 1] * rad)
    gam = ang_ref[0, 2] * rad
    cg = jnp.cos(gam)
    sg = jnp.clip(jnp.sin(gam), 1e-6, None)
    a, b, c = len_ref[0, 0], len_ref[0, 1], len_ref[0, 2]
    cy = (ca - cb_ * cg) / sg
    cz = jnp.sqrt(jnp.clip(1.0 - cb_ ** 2 - cy ** 2, 1e-6, None))
    cf2 = jnp.concatenate([a, b * cg, b * sg, c * cb_, c * cy, c * cz],
                          axis=0)

    Sb = S.astype(jnp.bfloat16)
    t1 = _dot3(fpk_ref[0], Sb)
    t1x, t1y, t1z = t1[0:GA], t1[GA:2 * GA], t1[2 * GA:3 * GA]

    def grow(x):
        return jnp.broadcast_to(x[:, None, :], (G, A, BA)).reshape(GA, BA)

    fr = fr_ref[0]
    t2x = grow(fr[0:G])
    t2y = grow(fr[G:2 * G])
    t2z = grow(fr[2 * G:3 * G])
    cfl = _dot3(cf2, Sb)
    l00 = grow(cfl[0:G])
    l10 = grow(cfl[G:2 * G])
    l11 = grow(cfl[2 * G:3 * G])
    l20 = grow(cfl[3 * G:4 * G])
    l21 = grow(cfl[4 * G:5 * G])
    l22 = grow(cfl[5 * G:6 * G])

    dx = t1x - t2x
    dx = dx - jnp.round(dx)
    dy = t1y - t2y
    dy = dy - jnp.round(dy)
    dz = t1z - t2z
    dz = dz - jnp.round(dz)
    cxx = dx * l00 + dy * l10 + dz * l20
    cyy = dy * l11 + dz * l21
    czz = dz * l22
    d2 = cxx * cxx + cyy * cyy + czz * czz + 1e-8
    inv_d = jax.lax.rsqrt(d2)
    dc = jnp.minimum(d2 * inv_d, CUT)

    env = 1.0 - dc * (1.0 / CUT)
    env = env * env * pmask_ref[:]

    e = jnp.exp(dc * dc * (-_INV2S))
    u = jnp.exp(dc * _UK)
    u2 = u * u
    u3 = u2 * u
    u4 = u2 * u2

    def quad(l, t):
        q = wrbf[l, 4 * t] + u * (wrbf[l, 4 * t + 1] * _K4M[t, 1])
        q = q + u2 * (wrbf[l, 4 * t + 2] * _K4M[t, 2])
        return q + u3 * (wrbf[l, 4 * t + 3] * _K4M[t, 3])

    w0 = e * quad(0, 0)
    w1 = e * quad(1, 0)
    for t in range(1, RBF // 4):
        e = (e * u4) * _K4C[t - 1]
        w0 = w0 + e * quad(0, t)
        w1 = w1 + e * quad(1, t)
    w0 = w0 * env
    w1 = w1 * env
    ux = cxx * inv_d
    uy = cyy * inv_d
    uz = czz * inv_d

    t = jnp.clip(types_ref[0, 0, :] - 1, 0, MAXZ - 1)
    oh = (t[:, None] == jax.lax.broadcasted_iota(jnp.int32, (CPS * A, MAXZ), 1)
          ).astype(f32)
    Hemb = jnp.dot(oh, emb_ref[:], preferred_element_type=f32)
    Hz = jnp.dot(z_ref[:], Wz_ref[:], preferred_element_type=f32) + bz_ref[:][None, :]
    H0 = Hemb + jnp.broadcast_to(Hz[:, None, :], (CPS, A, HID)).reshape(CPS * A, HID)

    W1w = W1_ref[:]
    b1w = b1_ref[:]
    wf = wf_ref[:]
    Watom = Watom_ref[:]
    batom = batom_ref[:]

    H = H0
    for l in range(LAYERS):
        w = w0 if l == 0 else w1
        ms = []
        for g in range(G):
            wl = w[g * A:(g + 1) * A]
            Wl = jnp.broadcast_to(wl[None], (CB, A, BA)).reshape(BA, BA) * bd
            ms.append(jnp.dot(Wl, H[g * BA:(g + 1) * BA],
                              preferred_element_type=f32))
        m = jnp.concatenate(ms, axis=0)
        H = H + jax.nn.relu(
            jnp.dot(m, W1w[l], preferred_element_type=f32) + b1w[l][None, :])

    logit_ref[:] = jnp.dot(H, Watom, preferred_element_type=f32) + batom[None, :]

    Hw = H * wf[None, :]
    spacks = []
    for g in range(G):
        blk = slice(g * BA, (g + 1) * BA)
        s = jax.lax.dot_general(Hw[blk], H[blk], (((1,), (1,)), ((), ())),
                                preferred_element_type=f32)
        s = s * bd
        spacks.append(s.reshape(CB, A, BA).sum(axis=0))

    spe = jnp.concatenate(spacks, axis=0) * env
    P = jnp.concatenate([spe * ux, spe * uy, spe * uz], axis=0)
    Fall = _dot3(P, Sb, (((1,), (1,)), ((), ())))
    for g in range(G):
        F_ref[g, 0] = Fall[g * A:(g + 1) * A]
        F_ref[g, 1] = Fall[GA + g * A:GA + (g + 1) * A]
        F_ref[g, 2] = Fall[2 * GA + g * A:2 * GA + (g + 1) * A]


def kernel(z, pred_frac_coords, pred_atom_types, num_atoms, lengths, angles,
           atom_emb, Wz, bz, w_rbf, W1, b1, w_f, W_atom, b_atom):
    del num_atoms
    frac6 = pred_frac_coords.reshape(NG, G, CB, A, 3)
    fpk = frac6.transpose(0, 4, 1, 3, 2).reshape(NG, 3 * GA, CB)
    fr = frac6.transpose(0, 4, 1, 2, 3).reshape(NG, 3 * G, BA)
    types3 = pred_atom_types.reshape(NG, 1, CPS * A)
    len4 = lengths.reshape(NG, G, CB, 3).transpose(0, 3, 1, 2)
    ang4 = angles.reshape(NG, G, CB, 3).transpose(0, 3, 1, 2)

    def rep(shape):
        return pl.BlockSpec(shape, lambda i: (0,) * len(shape))

    F, logits = pl.pallas_call(
        _block_kernel,
        grid=(NG,),
        in_specs=[
            pl.BlockSpec((CPS, LAT), lambda i: (i, 0)),
            pl.BlockSpec((1, 3 * GA, CB), lambda i: (i, 0, 0)),
            pl.BlockSpec((1, 3 * G, BA), lambda i: (i, 0, 0)),
            pl.BlockSpec((1, 1, CPS * A), lambda i: (i, 0, 0)),
            pl.BlockSpec((1, 3, G, CB), lambda i: (i, 0, 0, 0)),
            pl.BlockSpec((1, 3, G, CB), lambda i: (i, 0, 0, 0)),
            rep((CB, BA)),
            rep((GA, G)),
            rep((BA, A)),
            rep((A, BA)),
            rep((GA, BA)),
            rep((BA, BA)),
            rep((MAXZ, HID)),
            rep((LAT, HID)),
            rep((HID,)),
            rep((LAYERS, RBF)),
            rep((LAYERS, HID, HID)),
            rep((LAYERS, HID)),
            rep((HID,)),
            rep((HID, MAXZ)),
            rep((MAXZ,)),
        ],
        out_specs=(pl.BlockSpec((G, 3, A, CB), lambda i: (i, 0, 0, 0)),
                   pl.BlockSpec((CPS * A, MAXZ), lambda i: (i, 0))),
        out_shape=(jax.ShapeDtypeStruct((NG * G, 3, A, CB), jnp.float32),
                   jax.ShapeDtypeStruct((N, MAXZ), jnp.float32)),
        compiler_params=pltpu.CompilerParams(
            dimension_semantics=("parallel",)),
    )(z, fpk, fr, types3, len4, ang4,
      jnp.asarray(_S), jnp.asarray(_EA), jnp.asarray(_E24),
      jnp.asarray(_E24T), jnp.asarray(_PMASK), jnp.asarray(_BD),
      atom_emb, Wz, bz, w_rbf, W1, b1, w_f, W_atom, b_atom)
    F = F.transpose(0, 3, 2, 1).reshape(N, 3)
    return (F, logits)


# speedup vs baseline: 4.0531x; 1.0472x over previous
import jax
import jax.numpy as jnp
import numpy as np
from jax.experimental import pallas as pl
from jax.experimental.pallas import tpu as pltpu

C = 2048
A = 24
N = C * A
HID = 128
LAT = 256
RBF = 16
CUT = 6.0
MAXZ = 100
LAYERS = 2

CB = 8
BA = CB * A
G = 16
GA = G * A
CPS = G * CB
NG = C // CPS

_SIG2 = (CUT / RBF) ** 2
_INV2S = 1.0 / (2.0 * _SIG2)
_DELTA = CUT / (RBF - 1)
_UK = _DELTA / _SIG2
_KR = np.exp(-(2.0 * np.arange(RBF - 1) + 1.0) * _DELTA ** 2 * _INV2S)
_K4M = np.ones((RBF // 4, 4))
for _t in range(RBF // 4):
    for _m in range(1, 4):
        _K4M[_t, _m] = _K4M[_t, _m - 1] * _KR[4 * _t + _m - 1]
_K4C = np.array([_K4M[_t, 3] * _KR[4 * _t + 3] for _t in range(RBF // 4 - 1)])

_HP = jax.lax.Precision.HIGHEST

_S = np.zeros((CB, BA), np.float32)
for _c in range(CB):
    _S[_c, _c * A:(_c + 1) * A] = 1.0
_EA = np.zeros((GA, G), np.float32)
for _g in range(G):
    _EA[_g * A:(_g + 1) * A, _g] = 1.0
_E24 = np.tile(np.eye(A, dtype=np.float32), (CB, 1))
_E24T = np.tile(np.eye(A, dtype=np.float32), (1, CB))
_PMASK = np.tile(1.0 - np.eye(A, dtype=np.float32), (G, CB))
_cid = np.arange(BA) // A
_BD = (_cid[:, None] == _cid[None, :]).astype(np.float32)


def _dot3(x, sel_bf16, dims=None):
    f32 = jnp.float32
    x1 = x.astype(jnp.bfloat16)
    r = x - x1.astype(f32)
    x2 = r.astype(jnp.bfloat16)
    x3 = (r - x2.astype(f32)).astype(jnp.bfloat16)
    if dims is None:
        return (jnp.dot(x1, sel_bf16, preferred_element_type=f32)
                + jnp.dot(x2, sel_bf16, preferred_element_type=f32)
                + jnp.dot(x3, sel_bf16, preferred_element_type=f32))
    return (jax.lax.dot_general(x1, sel_bf16, dims, preferred_element_type=f32)
            + jax.lax.dot_general(x2, sel_bf16, dims, preferred_element_type=f32)
            + jax.lax.dot_general(x3, sel_bf16, dims, preferred_element_type=f32))


def _block_kernel(z_ref, fpk_ref, fr_ref, types_ref, len_ref, ang_ref,
                  S_ref, EA_ref, E24_ref, E24T_ref, pmask_ref, bd_ref,
                  emb_ref, Wz_ref, bz_ref, wrbf_ref, W1_ref, b1_ref,
                  wf_ref, Watom_ref, batom_ref, F_ref, logit_ref):
    f32 = jnp.float32
    S = S_ref[:]
    EA = EA_ref[:]
    E24 = E24_ref[:]
    E24T = E24T_ref[:]
    bd = bd_ref[:]
    wrbf = wrbf_ref[:]

    rad = np.pi / 180.0
    ca = jnp.cos(ang_ref[0, 0] * rad)
    cb_ = jnp.cos(ang_ref[0, 1] * rad)
    gam = ang_ref[0, 2] * rad
    cg = jnp.cos(gam)
    sg = jnp.clip(jnp.sin(gam), 1e-6, None)
    a, b, c = len_ref[0, 0], len_ref[0, 1], len_ref[0, 2]
    cy = (ca - cb_ * cg) / sg
    cz = jnp.sqrt(jnp.clip(1.0 - cb_ ** 2 - cy ** 2, 1e-6, None))
    cf2 = jnp.concatenate([a, b * cg, b * sg, c * cb_, c * cy, c * cz],
                          axis=0)

    Sb = S.astype(jnp.bfloat16)
    t1 = _dot3(fpk_ref[0], Sb)
    t1x, t1y, t1z = t1[0:GA], t1[GA:2 * GA], t1[2 * GA:3 * GA]

    def grow(x):
        return jnp.broadcast_to(x[:, None, :], (G, A, BA)).reshape(GA, BA)

    fr = fr_ref[0]
    t2x = grow(fr[0:G])
    t2y = grow(fr[G:2 * G])
    t2z = grow(fr[2 * G:3 * G])
    cfl = _dot3(cf2, Sb)
    l00 = grow(cfl[0:G])
    l10 = grow(cfl[G:2 * G])
    l11 = grow(cfl[2 * G:3 * G])
    l20 = grow(cfl[3 * G:4 * G])
    l21 = grow(cfl[4 * G:5 * G])
    l22 = grow(cfl[5 * G:6 * G])

    dx = t1x - t2x
    dx = dx - jnp.round(dx)
    dy = t1y - t2y
    dy = dy - jnp.round(dy)
    dz = t1z - t2z
    dz = dz - jnp.round(dz)
    cxx = dx * l00 + dy * l10 + dz * l20
    cyy = dy * l11 + dz * l21
    czz = dz * l22
    d2 = cxx * cxx + cyy * cyy + czz * czz + 1e-8
    inv_d = jax.lax.rsqrt(d2)
    dc = jnp.minimum(d2 * inv_d, CUT)

    env = 1.0 - dc * (1.0 / CUT)
    env = env * env * pmask_ref[:]

    e = jnp.exp(dc * dc * (-_INV2S))
    u = jnp.exp(dc * _UK)
    u2 = u * u
    u3 = u2 * u
    u4 = u2 * u2

    def quad(l, t):
        q = wrbf[l, 4 * t] + u * (wrbf[l, 4 * t + 1] * _K4M[t, 1])
        q = q + u2 * (wrbf[l, 4 * t + 2] * _K4M[t, 2])
        return q + u3 * (wrbf[l, 4 * t + 3] * _K4M[t, 3])

    w0 = e * quad(0, 0)
    w1 = e * quad(1, 0)
    for t in range(1, RBF // 4):
        e = (e * u4) * _K4C[t - 1]
        w0 = w0 + e * quad(0, t)
        w1 = w1 + e * quad(1, t)
    w0 = w0 * env
    w1 = w1 * env
    ux = cxx * inv_d
    uy = cyy * inv_d
    uz = czz * inv_d

    t = jnp.clip(types_ref[0, 0, :] - 1, 0, MAXZ - 1)
    oh = (t[:, None] == jax.lax.broadcasted_iota(jnp.int32, (CPS * A, MAXZ), 1)
          ).astype(f32)
    Hemb = jnp.dot(oh, emb_ref[:], preferred_element_type=f32)
    Hz = jnp.dot(z_ref[:], Wz_ref[:], preferred_element_type=f32) + bz_ref[:][None, :]
    H0 = Hemb + jnp.broadcast_to(Hz[:, None, :], (CPS, A, HID)).reshape(CPS * A, HID)

    W1w = W1_ref[:]
    b1w = b1_ref[:]
    wf = wf_ref[:]
    Watom = Watom_ref[:]
    batom = batom_ref[:]

    H = H0
    for l in range(LAYERS):
        w = w0 if l == 0 else w1
        ms = []
        for g in range(G):
            wl = w[g * A:(g + 1) * A]
            Wl = jnp.broadcast_to(wl[None], (CB, A, BA)).reshape(BA, BA) * bd
            ms.append(jnp.dot(Wl, H[g * BA:(g + 1) * BA],
                              preferred_element_type=f32))
        m = jnp.concatenate(ms, axis=0)
        H = H + jax.nn.relu(
            jnp.dot(m, W1w[l], preferred_element_type=f32) + b1w[l][None, :])

    logit_ref[:] = jnp.dot(H, Watom, preferred_element_type=f32) + batom[None, :]

    Hw = H * wf[None, :]
    spacks = []
    for g in range(G):
        blk = slice(g * BA, (g + 1) * BA)
        s = jax.lax.dot_general(Hw[blk], H[blk], (((1,), (1,)), ((), ())),
                                preferred_element_type=f32)
        s = s * bd
        spacks.append(s.reshape(CB, A, BA).sum(axis=0))

    spe = jnp.concatenate(spacks, axis=0) * env
    P = jnp.concatenate([spe * ux, spe * uy, spe * uz], axis=0)
    Fall = _dot3(P, Sb, (((1,), (1,)), ((), ())))
    for g in range(G):
        F_ref[g, 0] = Fall[g * A:(g + 1) * A]
        F_ref[g, 1] = Fall[GA + g * A:GA + (g + 1) * A]
        F_ref[g, 2] = Fall[2 * GA + g * A:2 * GA + (g + 1) * A]


def kernel(z, pred_frac_coords, pred_atom_types, num_atoms, lengths, angles,
           atom_emb, Wz, bz, w_rbf, W1, b1, w_f, W_atom, b_atom):
    del num_atoms
    frac6 = pred_frac_coords.reshape(NG, G, CB, A, 3)
    fpk = frac6.transpose(0, 4, 1, 3, 2).reshape(NG, 3 * GA, CB)
    fr = frac6.transpose(0, 4, 1, 2, 3).reshape(NG, 3 * G, BA)
    types3 = pred_atom_types.reshape(NG, 1, CPS * A)
    len4 = lengths.reshape(NG, G, CB, 3).transpose(0, 3, 1, 2)
    ang4 = angles.reshape(NG, G, CB, 3).transpose(0, 3, 1, 2)

    def rep(shape):
        return pl.BlockSpec(shape, lambda i: (0,) * len(shape))

    F, logits = pl.pallas_call(
        _block_kernel,
        grid=(NG,),
        in_specs=[
            pl.BlockSpec((CPS, LAT), lambda i: (i, 0)),
            pl.BlockSpec((1, 3 * GA, CB), lambda i: (i, 0, 0)),
            pl.BlockSpec((1, 3 * G, BA), lambda i: (i, 0, 0)),
            pl.BlockSpec((1, 1, CPS * A), lambda i: (i, 0, 0)),
            pl.BlockSpec((1, 3, G, CB), lambda i: (i, 0, 0, 0)),
            pl.BlockSpec((1, 3, G, CB), lambda i: (i, 0, 0, 0)),
            rep((CB, BA)),
            rep((GA, G)),
            rep((BA, A)),
            rep((A, BA)),
            rep((GA, BA)),
            rep((BA, BA)),
            rep((MAXZ, HID)),
            rep((LAT, HID)),
            rep((HID,)),
            rep((LAYERS, RBF)),
            rep((LAYERS, HID, HID)),
            rep((LAYERS, HID)),
            rep((HID,)),
            rep((HID, MAXZ)),
            rep((MAXZ,)),
        ],
        out_specs=(pl.BlockSpec((G, 3, A, CB), lambda i: (i, 0, 0, 0)),
                   pl.BlockSpec((CPS * A, MAXZ), lambda i: (i, 0))),
        out_shape=(jax.ShapeDtypeStruct((NG * G, 3, A, CB), jnp.float32),
                   jax.ShapeDtypeStruct((N, MAXZ), jnp.float32)),
        compiler_params=pltpu.CompilerParams(
            dimension_semantics=("parallel",)),
    )(z, fpk, fr, types3, len4, ang4,
      jnp.asarray(_S), jnp.asarray(_EA), jnp.asarray(_E24),
      jnp.asarray(_E24T), jnp.asarray(_PMASK), jnp.asarray(_BD),
      atom_emb, Wz, bz, w_rbf, W1, b1, w_f, W_atom, b_atom)
    F = F.transpose(0, 3, 2, 1).reshape(N, 3)
    return (F, logits)


# speedup vs baseline: 4.0852x; 1.0079x over previous
import jax
import jax.numpy as jnp
import numpy as np
from jax.experimental import pallas as pl
from jax.experimental.pallas import tpu as pltpu

C = 2048
A = 24
N = C * A
HID = 128
LAT = 256
RBF = 16
CUT = 6.0
MAXZ = 100
LAYERS = 2

CB = 8
BA = CB * A
G = 32
GA = G * A
CPS = G * CB
NG = C // CPS

_SIG2 = (CUT / RBF) ** 2
_INV2S = 1.0 / (2.0 * _SIG2)
_DELTA = CUT / (RBF - 1)
_UK = _DELTA / _SIG2
_KR = np.exp(-(2.0 * np.arange(RBF - 1) + 1.0) * _DELTA ** 2 * _INV2S)
_K4M = np.ones((RBF // 4, 4))
for _t in range(RBF // 4):
    for _m in range(1, 4):
        _K4M[_t, _m] = _K4M[_t, _m - 1] * _KR[4 * _t + _m - 1]
_K4C = np.array([_K4M[_t, 3] * _KR[4 * _t + 3] for _t in range(RBF // 4 - 1)])

_HP = jax.lax.Precision.HIGHEST

_S = np.zeros((CB, BA), np.float32)
for _c in range(CB):
    _S[_c, _c * A:(_c + 1) * A] = 1.0
_EA = np.zeros((GA, G), np.float32)
for _g in range(G):
    _EA[_g * A:(_g + 1) * A, _g] = 1.0
_E24 = np.tile(np.eye(A, dtype=np.float32), (CB, 1))
_E24T = np.tile(np.eye(A, dtype=np.float32), (1, CB))
_PMASK = np.tile(1.0 - np.eye(A, dtype=np.float32), (G, CB))
_cid = np.arange(BA) // A
_BD = (_cid[:, None] == _cid[None, :]).astype(np.float32)


def _dot3(x, sel_bf16, dims=None):
    f32 = jnp.float32
    x1 = x.astype(jnp.bfloat16)
    r = x - x1.astype(f32)
    x2 = r.astype(jnp.bfloat16)
    x3 = (r - x2.astype(f32)).astype(jnp.bfloat16)
    if dims is None:
        return (jnp.dot(x1, sel_bf16, preferred_element_type=f32)
                + jnp.dot(x2, sel_bf16, preferred_element_type=f32)
                + jnp.dot(x3, sel_bf16, preferred_element_type=f32))
    return (jax.lax.dot_general(x1, sel_bf16, dims, preferred_element_type=f32)
            + jax.lax.dot_general(x2, sel_bf16, dims, preferred_element_type=f32)
            + jax.lax.dot_general(x3, sel_bf16, dims, preferred_element_type=f32))


def _block_kernel(z_ref, fpk_ref, fr_ref, types_ref, len_ref, ang_ref,
                  S_ref, EA_ref, E24_ref, E24T_ref, pmask_ref, bd_ref,
                  emb_ref, Wz_ref, bz_ref, wrbf_ref, W1_ref, b1_ref,
                  wf_ref, Watom_ref, batom_ref, F_ref, logit_ref):
    f32 = jnp.float32
    S = S_ref[:]
    EA = EA_ref[:]
    E24 = E24_ref[:]
    E24T = E24T_ref[:]
    bd = bd_ref[:]
    wrbf = wrbf_ref[:]

    rad = np.pi / 180.0
    ca = jnp.cos(ang_ref[0, 0] * rad)
    cb_ = jnp.cos(ang_ref[0, 1] * rad)
    gam = ang_ref[0, 2] * rad
    cg = jnp.cos(gam)
    sg = jnp.clip(jnp.sin(gam), 1e-6, None)
    a, b, c = len_ref[0, 0], len_ref[0, 1], len_ref[0, 2]
    cy = (ca - cb_ * cg) / sg
    cz = jnp.sqrt(jnp.clip(1.0 - cb_ ** 2 - cy ** 2, 1e-6, None))
    cf2 = jnp.concatenate([a, b * cg, b * sg, c * cb_, c * cy, c * cz],
                          axis=0)

    Sb = S.astype(jnp.bfloat16)
    t1 = _dot3(fpk_ref[0], Sb)
    t1x, t1y, t1z = t1[0:GA], t1[GA:2 * GA], t1[2 * GA:3 * GA]

    def grow(x):
        return jnp.broadcast_to(x[:, None, :], (G, A, BA)).reshape(GA, BA)

    fr = fr_ref[0]
    t2x = grow(fr[0:G])
    t2y = grow(fr[G:2 * G])
    t2z = grow(fr[2 * G:3 * G])
    cfl = _dot3(cf2, Sb)
    l00 = grow(cfl[0:G])
    l10 = grow(cfl[G:2 * G])
    l11 = grow(cfl[2 * G:3 * G])
    l20 = grow(cfl[3 * G:4 * G])
    l21 = grow(cfl[4 * G:5 * G])
    l22 = grow(cfl[5 * G:6 * G])

    dx = t1x - t2x
    dx = dx - jnp.round(dx)
    dy = t1y - t2y
    dy = dy - jnp.round(dy)
    dz = t1z - t2z
    dz = dz - jnp.round(dz)
    cxx = dx * l00 + dy * l10 + dz * l20
    cyy = dy * l11 + dz * l21
    czz = dz * l22
    d2 = cxx * cxx + cyy * cyy + czz * czz + 1e-8
    inv_d = jax.lax.rsqrt(d2)
    dc = jnp.minimum(d2 * inv_d, CUT)

    env = 1.0 - dc * (1.0 / CUT)
    env = env * env * pmask_ref[:]

    e = jnp.exp(dc * dc * (-_INV2S))
    u = jnp.exp(dc * _UK)
    u2 = u * u
    u3 = u2 * u
    u4 = u2 * u2

    def quad(l, t):
        q = wrbf[l, 4 * t] + u * (wrbf[l, 4 * t + 1] * _K4M[t, 1])
        q = q + u2 * (wrbf[l, 4 * t + 2] * _K4M[t, 2])
        return q + u3 * (wrbf[l, 4 * t + 3] * _K4M[t, 3])

    w0 = e * quad(0, 0)
    w1 = e * quad(1, 0)
    for t in range(1, RBF // 4):
        e = (e * u4) * _K4C[t - 1]
        w0 = w0 + e * quad(0, t)
        w1 = w1 + e * quad(1, t)
    w0 = w0 * env
    w1 = w1 * env
    ux = cxx * inv_d
    uy = cyy * inv_d
    uz = czz * inv_d

    t = jnp.clip(types_ref[0, 0, :] - 1, 0, MAXZ - 1)
    oh = (t[:, None] == jax.lax.broadcasted_iota(jnp.int32, (CPS * A, MAXZ), 1)
          ).astype(f32)
    Hemb = jnp.dot(oh, emb_ref[:], preferred_element_type=f32)
    Hz = jnp.dot(z_ref[:], Wz_ref[:], preferred_element_type=f32) + bz_ref[:][None, :]
    H0 = Hemb + jnp.broadcast_to(Hz[:, None, :], (CPS, A, HID)).reshape(CPS * A, HID)

    W1w = W1_ref[:]
    b1w = b1_ref[:]
    wf = wf_ref[:]
    Watom = Watom_ref[:]
    batom = batom_ref[:]

    H = H0
    for l in range(LAYERS):
        w = w0 if l == 0 else w1
        ms = []
        for g in range(G):
            wl = w[g * A:(g + 1) * A]
            Wl = jnp.broadcast_to(wl[None], (CB, A, BA)).reshape(BA, BA) * bd
            ms.append(jnp.dot(Wl, H[g * BA:(g + 1) * BA],
                              preferred_element_type=f32))
        m = jnp.concatenate(ms, axis=0)
        H = H + jax.nn.relu(
            jnp.dot(m, W1w[l], preferred_element_type=f32) + b1w[l][None, :])

    logit_ref[:] = jnp.dot(H, Watom, preferred_element_type=f32) + batom[None, :]

    Hw = H * wf[None, :]
    spacks = []
    for g in range(G):
        blk = slice(g * BA, (g + 1) * BA)
        s = jax.lax.dot_general(Hw[blk], H[blk], (((1,), (1,)), ((), ())),
                                preferred_element_type=f32)
        s = s * bd
        spacks.append(s.reshape(CB, A, BA).sum(axis=0))

    spe = jnp.concatenate(spacks, axis=0) * env
    P = jnp.concatenate([spe * ux, spe * uy, spe * uz], axis=0)
    Fall = _dot3(P, Sb, (((1,), (1,)), ((), ())))
    for g in range(G):
        F_ref[g, 0] = Fall[g * A:(g + 1) * A]
        F_ref[g, 1] = Fall[GA + g * A:GA + (g + 1) * A]
        F_ref[g, 2] = Fall[2 * GA + g * A:2 * GA + (g + 1) * A]


def kernel(z, pred_frac_coords, pred_atom_types, num_atoms, lengths, angles,
           atom_emb, Wz, bz, w_rbf, W1, b1, w_f, W_atom, b_atom):
    del num_atoms
    frac6 = pred_frac_coords.reshape(NG, G, CB, A, 3)
    fpk = frac6.transpose(0, 4, 1, 3, 2).reshape(NG, 3 * GA, CB)
    fr = frac6.transpose(0, 4, 1, 2, 3).reshape(NG, 3 * G, BA)
    types3 = pred_atom_types.reshape(NG, 1, CPS * A)
    len4 = lengths.reshape(NG, G, CB, 3).transpose(0, 3, 1, 2)
    ang4 = angles.reshape(NG, G, CB, 3).transpose(0, 3, 1, 2)

    def rep(shape):
        return pl.BlockSpec(shape, lambda i: (0,) * len(shape))

    F, logits = pl.pallas_call(
        _block_kernel,
        grid=(NG,),
        in_specs=[
            pl.BlockSpec((CPS, LAT), lambda i: (i, 0)),
            pl.BlockSpec((1, 3 * GA, CB), lambda i: (i, 0, 0)),
            pl.BlockSpec((1, 3 * G, BA), lambda i: (i, 0, 0)),
            pl.BlockSpec((1, 1, CPS * A), lambda i: (i, 0, 0)),
            pl.BlockSpec((1, 3, G, CB), lambda i: (i, 0, 0, 0)),
            pl.BlockSpec((1, 3, G, CB), lambda i: (i, 0, 0, 0)),
            rep((CB, BA)),
            rep((GA, G)),
            rep((BA, A)),
            rep((A, BA)),
            rep((GA, BA)),
            rep((BA, BA)),
            rep((MAXZ, HID)),
            rep((LAT, HID)),
            rep((HID,)),
            rep((LAYERS, RBF)),
            rep((LAYERS, HID, HID)),
            rep((LAYERS, HID)),
            rep((HID,)),
            rep((HID, MAXZ)),
            rep((MAXZ,)),
        ],
        out_specs=(pl.BlockSpec((G, 3, A, CB), lambda i: (i, 0, 0, 0)),
                   pl.BlockSpec((CPS * A, MAXZ), lambda i: (i, 0))),
        out_shape=(jax.ShapeDtypeStruct((NG * G, 3, A, CB), jnp.float32),
                   jax.ShapeDtypeStruct((N, MAXZ), jnp.float32)),
        compiler_params=pltpu.CompilerParams(
            dimension_semantics=("parallel",)),
    )(z, fpk, fr, types3, len4, ang4,
      jnp.asarray(_S), jnp.asarray(_EA), jnp.asarray(_E24),
      jnp.asarray(_E24T), jnp.asarray(_PMASK), jnp.asarray(_BD),
      atom_emb, Wz, bz, w_rbf, W1, b1, w_f, W_atom, b_atom)
    F = F.transpose(0, 3, 2, 1).reshape(N, 3)
    return (F, logits)


# speedup vs baseline: 4.0858x; 1.0001x over previous
import jax
import jax.numpy as jnp
import numpy as np
from jax.experimental import pallas as pl
from jax.experimental.pallas import tpu as pltpu

C = 2048
A = 24
N = C * A
HID = 128
LAT = 256
RBF = 16
CUT = 6.0
MAXZ = 100
LAYERS = 2

CB = 8
BA = CB * A
G = 32
GA = G * A
CPS = G * CB
NG = C // CPS

_SIG2 = (CUT / RBF) ** 2
_INV2S = 1.0 / (2.0 * _SIG2)
_DELTA = CUT / (RBF - 1)
_UK = _DELTA / _SIG2
_KR = np.exp(-(2.0 * np.arange(RBF - 1) + 1.0) * _DELTA ** 2 * _INV2S)
_K4M = np.ones((RBF // 4, 4))
for _t in range(RBF // 4):
    for _m in range(1, 4):
        _K4M[_t, _m] = _K4M[_t, _m - 1] * _KR[4 * _t + _m - 1]
_K4C = np.array([_K4M[_t, 3] * _KR[4 * _t + 3] for _t in range(RBF // 4 - 1)])


_S = np.zeros((CB, BA), np.float32)
for _c in range(CB):
    _S[_c, _c * A:(_c + 1) * A] = 1.0
_PMASK = np.tile(1.0 - np.eye(A, dtype=np.float32), (G, CB))
_cid = np.arange(BA) // A
_BD = (_cid[:, None] == _cid[None, :]).astype(np.float32)


def _dot3(x, sel_bf16, dims=None):
    f32 = jnp.float32
    x1 = x.astype(jnp.bfloat16)
    r = x - x1.astype(f32)
    x2 = r.astype(jnp.bfloat16)
    x3 = (r - x2.astype(f32)).astype(jnp.bfloat16)
    if dims is None:
        return (jnp.dot(x1, sel_bf16, preferred_element_type=f32)
                + jnp.dot(x2, sel_bf16, preferred_element_type=f32)
                + jnp.dot(x3, sel_bf16, preferred_element_type=f32))
    return (jax.lax.dot_general(x1, sel_bf16, dims, preferred_element_type=f32)
            + jax.lax.dot_general(x2, sel_bf16, dims, preferred_element_type=f32)
            + jax.lax.dot_general(x3, sel_bf16, dims, preferred_element_type=f32))


def _block_kernel(z_ref, fpk_ref, fr_ref, types_ref, len_ref, ang_ref,
                  S_ref, pmask_ref, bd_ref,
                  emb_ref, Wz_ref, bz_ref, wrbf_ref, W1_ref, b1_ref,
                  wf_ref, Watom_ref, batom_ref, F_ref, logit_ref):
    f32 = jnp.float32
    S = S_ref[:]
    bd = bd_ref[:]
    wrbf = wrbf_ref[:]

    rad = np.pi / 180.0
    ca = jnp.cos(ang_ref[0, 0] * rad)
    cb_ = jnp.cos(ang_ref[0, 1] * rad)
    gam = ang_ref[0, 2] * rad
    cg = jnp.cos(gam)
    sg = jnp.clip(jnp.sin(gam), 1e-6, None)
    a, b, c = len_ref[0, 0], len_ref[0, 1], len_ref[0, 2]
    cy = (ca - cb_ * cg) / sg
    cz = jnp.sqrt(jnp.clip(1.0 - cb_ ** 2 - cy ** 2, 1e-6, None))
    cf2 = jnp.concatenate([a, b * cg, b * sg, c * cb_, c * cy, c * cz],
                          axis=0)

    Sb = S.astype(jnp.bfloat16)
    t1 = _dot3(fpk_ref[0], Sb)
    t1x, t1y, t1z = t1[0:GA], t1[GA:2 * GA], t1[2 * GA:3 * GA]

    def grow(x):
        return jnp.broadcast_to(x[:, None, :], (G, A, BA)).reshape(GA, BA)

    fr = fr_ref[0]
    t2x = grow(fr[0:G])
    t2y = grow(fr[G:2 * G])
    t2z = grow(fr[2 * G:3 * G])
    cfl = _dot3(cf2, Sb)
    l00 = grow(cfl[0:G])
    l10 = grow(cfl[G:2 * G])
    l11 = grow(cfl[2 * G:3 * G])
    l20 = grow(cfl[3 * G:4 * G])
    l21 = grow(cfl[4 * G:5 * G])
    l22 = grow(cfl[5 * G:6 * G])

    dx = t1x - t2x
    dx = dx - jnp.round(dx)
    dy = t1y - t2y
    dy = dy - jnp.round(dy)
    dz = t1z - t2z
    dz = dz - jnp.round(dz)
    cxx = dx * l00 + dy * l10 + dz * l20
    cyy = dy * l11 + dz * l21
    czz = dz * l22
    d2 = cxx * cxx + cyy * cyy + czz * czz + 1e-8
    inv_d = jax.lax.rsqrt(d2)
    dc = jnp.minimum(d2 * inv_d, CUT)

    env = 1.0 - dc * (1.0 / CUT)
    env = env * env * pmask_ref[:]

    e = jnp.exp(dc * dc * (-_INV2S))
    u = jnp.exp(dc * _UK)
    u2 = u * u
    u3 = u2 * u
    u4 = u2 * u2

    def quad(l, t):
        q = wrbf[l, 4 * t] + u * (wrbf[l, 4 * t + 1] * _K4M[t, 1])
        q = q + u2 * (wrbf[l, 4 * t + 2] * _K4M[t, 2])
        return q + u3 * (wrbf[l, 4 * t + 3] * _K4M[t, 3])

    w0 = e * quad(0, 0)
    w1 = e * quad(1, 0)
    for t in range(1, RBF // 4):
        e = (e * u4) * _K4C[t - 1]
        w0 = w0 + e * quad(0, t)
        w1 = w1 + e * quad(1, t)
    w0 = w0 * env
    w1 = w1 * env
    ux = cxx * inv_d
    uy = cyy * inv_d
    uz = czz * inv_d

    t = jnp.clip(types_ref[0, 0, :] - 1, 0, MAXZ - 1)
    oh = (t[:, None] == jax.lax.broadcasted_iota(jnp.int32, (CPS * A, MAXZ), 1)
          ).astype(f32)
    Hemb = jnp.dot(oh, emb_ref[:], preferred_element_type=f32)
    Hz = jnp.dot(z_ref[:], Wz_ref[:], preferred_element_type=f32) + bz_ref[:][None, :]
    H0 = Hemb + jnp.broadcast_to(Hz[:, None, :], (CPS, A, HID)).reshape(CPS * A, HID)

    W1w = W1_ref[:]
    b1w = b1_ref[:]
    wf = wf_ref[:]
    Watom = Watom_ref[:]
    batom = batom_ref[:]

    H = H0
    for l in range(LAYERS):
        w = w0 if l == 0 else w1
        ms = []
        for g in range(G):
            wl = w[g * A:(g + 1) * A]
            Wl = jnp.broadcast_to(wl[None], (CB, A, BA)).reshape(BA, BA) * bd
            ms.append(jnp.dot(Wl, H[g * BA:(g + 1) * BA],
                              preferred_element_type=f32))
        m = jnp.concatenate(ms, axis=0)
        H = H + jax.nn.relu(
            jnp.dot(m, W1w[l], preferred_element_type=f32) + b1w[l][None, :])

    logit_ref[:] = jnp.dot(H, Watom, preferred_element_type=f32) + batom[None, :]

    Hw = H * wf[None, :]
    spacks = []
    for g in range(G):
        blk = slice(g * BA, (g + 1) * BA)
        s = jax.lax.dot_general(Hw[blk], H[blk], (((1,), (1,)), ((), ())),
                                preferred_element_type=f32)
        s = s * bd
        spacks.append(s.reshape(CB, A, BA).sum(axis=0))

    spe = jnp.concatenate(spacks, axis=0) * env
    P = jnp.concatenate([spe * ux, spe * uy, spe * uz], axis=0)
    Fall = _dot3(P, Sb, (((1,), (1,)), ((), ())))
    for g in range(G):
        F_ref[g, 0] = Fall[g * A:(g + 1) * A]
        F_ref[g, 1] = Fall[GA + g * A:GA + (g + 1) * A]
        F_ref[g, 2] = Fall[2 * GA + g * A:2 * GA + (g + 1) * A]


def kernel(z, pred_frac_coords, pred_atom_types, num_atoms, lengths, angles,
           atom_emb, Wz, bz, w_rbf, W1, b1, w_f, W_atom, b_atom):
    del num_atoms
    frac6 = pred_frac_coords.reshape(NG, G, CB, A, 3)
    fpk = frac6.transpose(0, 4, 1, 3, 2).reshape(NG, 3 * GA, CB)
    fr = frac6.transpose(0, 4, 1, 2, 3).reshape(NG, 3 * G, BA)
    types3 = pred_atom_types.reshape(NG, 1, CPS * A)
    len4 = lengths.reshape(NG, G, CB, 3).transpose(0, 3, 1, 2)
    ang4 = angles.reshape(NG, G, CB, 3).transpose(0, 3, 1, 2)

    def rep(shape):
        return pl.BlockSpec(shape, lambda i: (0,) * len(shape))

    F, logits = pl.pallas_call(
        _block_kernel,
        grid=(NG,),
        in_specs=[
            pl.BlockSpec((CPS, LAT), lambda i: (i, 0)),
            pl.BlockSpec((1, 3 * GA, CB), lambda i: (i, 0, 0)),
            pl.BlockSpec((1, 3 * G, BA), lambda i: (i, 0, 0)),
            pl.BlockSpec((1, 1, CPS * A), lambda i: (i, 0, 0)),
            pl.BlockSpec((1, 3, G, CB), lambda i: (i, 0, 0, 0)),
            pl.BlockSpec((1, 3, G, CB), lambda i: (i, 0, 0, 0)),
            rep((CB, BA)),
            rep((GA, BA)),
            rep((BA, BA)),
            rep((MAXZ, HID)),
            rep((LAT, HID)),
            rep((HID,)),
            rep((LAYERS, RBF)),
            rep((LAYERS, HID, HID)),
            rep((LAYERS, HID)),
            rep((HID,)),
            rep((HID, MAXZ)),
            rep((MAXZ,)),
        ],
        out_specs=(pl.BlockSpec((G, 3, A, CB), lambda i: (i, 0, 0, 0)),
                   pl.BlockSpec((CPS * A, MAXZ), lambda i: (i, 0))),
        out_shape=(jax.ShapeDtypeStruct((NG * G, 3, A, CB), jnp.float32),
                   jax.ShapeDtypeStruct((N, MAXZ), jnp.float32)),
        compiler_params=pltpu.CompilerParams(
            dimension_semantics=("parallel",)),
    )(z, fpk, fr, types3, len4, ang4,
      jnp.asarray(_S), jnp.asarray(_PMASK), jnp.asarray(_BD),
      atom_emb, Wz, bz, w_rbf, W1, b1, w_f, W_atom, b_atom)
    F = F.transpose(0, 3, 2, 1).reshape(N, 3)
    return (F, logits)
